# Initial kernel scaffold; baseline (speedup 1.0000x reference)
import functools

import jax
import jax.numpy as jnp
from jax import lax
from jax.experimental import pallas as pl
from jax.experimental.pallas import tpu as pltpu
from jax.experimental.pallas import tpu_sc as plsc

N = 10000
E = 320000
D = 128
K = 3
NP = 10240
NC = 2
NS = 16
NW = NC * NS
EW = E // NW
CE = 80
NCH = EW // CE
RPW = NP // NS

_mesh = plsc.VectorSubcoreMesh(core_axis_name="c", subcore_axis_name="s")


def _proj_body(x_ref, ws_ref, wd_ref, wa_ref, b_ref, fs_ref, fd_ref, at_ref,
               cm_ref):
    xb = x_ref[...]
    dn = (((1,), (1,)), ((), ()))
    fs_ref[...] = lax.dot_general(xb, ws_ref[...], dn,
                                  preferred_element_type=jnp.float32)
    fd_ref[...] = lax.dot_general(xb, wd_ref[...], dn,
                                  preferred_element_type=jnp.float32) + b_ref[...]
    at = lax.dot_general(xb, wa_ref[...], dn,
                         preferred_element_type=jnp.float32)
    at_ref[...] = at
    m8 = jnp.broadcast_to(jnp.max(at, axis=0, keepdims=True), (8, 128))

    @pl.when(pl.program_id(0) == 0)
    def _():
        cm_ref[...] = m8

    @pl.when(pl.program_id(0) > 0)
    def _():
        cm_ref[...] = jnp.maximum(cm_ref[...], m8)


def _proj(x, w_src, w_dst, wa_pad, b_row):
    bp = 1000
    return pl.pallas_call(
        _proj_body,
        grid=(N // bp,),
        in_specs=[
            pl.BlockSpec((bp, 128), lambda i: (i, 0)),
            pl.BlockSpec((128, 128), lambda i: (0, 0)),
            pl.BlockSpec((128, 128), lambda i: (0, 0)),
            pl.BlockSpec((128, 128), lambda i: (0, 0)),
            pl.BlockSpec((1, 128), lambda i: (0, 0)),
        ],
        out_specs=[
            pl.BlockSpec((bp, 128), lambda i: (i, 0)),
            pl.BlockSpec((bp, 128), lambda i: (i, 0)),
            pl.BlockSpec((bp, 128), lambda i: (i, 0)),
            pl.BlockSpec((8, 128), lambda i: (0, 0)),
        ],
        out_shape=[
            jax.ShapeDtypeStruct((N, 128), jnp.float32),
            jax.ShapeDtypeStruct((N, 128), jnp.float32),
            jax.ShapeDtypeStruct((N, 128), jnp.float32),
            jax.ShapeDtypeStruct((8, 128), jnp.float32),
        ],
    )(x, w_src, w_dst, wa_pad, b_row)


def _stats_body(asrc_hbm, adst_hbm, srci_hbm, dsti_hbm, cvec_hbm,
                ex_hbm, sd_hbm, ss_hbm,
                idx_s, idx_d, vas, vad, vex, cv_v, szero, sd_sh, ss_sh,
                sem_a, sem_b):
    cid = lax.axis_index("c")
    sid = lax.axis_index("s")
    wid = cid * NS + sid

    pltpu.sync_copy(cvec_hbm, cv_v)
    cv = cv_v[...]

    def _z(i, _):
        szero[pl.ds(i * 16, 16)] = jnp.zeros((16,), jnp.float32)
        return 0
    lax.fori_loop(0, RPW // 16, _z, 0)
    pltpu.sync_copy(szero, sd_sh.at[pl.ds(sid * RPW, RPW)])
    pltpu.sync_copy(szero, ss_sh.at[pl.ds(sid * RPW, RPW)])
    plsc.subcore_barrier()

    def _chunk(j, _):
        base = wid * EW + j * CE
        pltpu.sync_copy(srci_hbm.at[pl.ds(base, CE)], idx_s)
        pltpu.sync_copy(dsti_hbm.at[pl.ds(base, CE)], idx_d)
        c1 = pltpu.async_copy(asrc_hbm.at[idx_s], vas, sem_a)
        c2 = pltpu.async_copy(adst_hbm.at[idx_d], vad, sem_b)
        c1.wait()
        c2.wait()
        for t in range(CE // 16):
            sl = pl.ds(t * 16, 16)
            v = vas[sl] + vad[sl]
            e = jnp.where(v >= 0.0, v, v * jnp.float32(0.2))
            vex[sl] = jnp.exp(e - cv)
        pltpu.sync_copy(vex, ex_hbm.at[pl.ds(base, CE)])
        pltpu.sync_copy(vex, sd_sh.at[idx_d], add=True)
        pltpu.sync_copy(vex, ss_sh.at[idx_s], add=True)
        return 0

    lax.fori_loop(0, NCH, _chunk, 0)
    plsc.subcore_barrier()
    sl = pl.ds(sid * RPW, RPW)
    pltpu.sync_copy(sd_sh.at[sl], sd_hbm.at[cid, sl])
    pltpu.sync_copy(ss_sh.at[sl], ss_hbm.at[cid, sl])


_stats_call = functools.partial(
    pl.kernel,
    out_type=(
        jax.ShapeDtypeStruct((E,), jnp.float32),
        jax.ShapeDtypeStruct((NC, NP), jnp.float32),
        jax.ShapeDtypeStruct((NC, NP), jnp.float32),
    ),
    mesh=_mesh,
    scratch_types=[
        pltpu.VMEM((CE,), jnp.int32),
        pltpu.VMEM((CE,), jnp.int32),
        pltpu.VMEM((CE,), jnp.float32),
        pltpu.VMEM((CE,), jnp.float32),
        pltpu.VMEM((CE,), jnp.float32),
        pltpu.VMEM((16,), jnp.float32),
        pltpu.VMEM((RPW,), jnp.float32),
        pltpu.VMEM_SHARED((NP,), jnp.float32),
        pltpu.VMEM_SHARED((NP,), jnp.float32),
        pltpu.SemaphoreType.DMA,
        pltpu.SemaphoreType.DMA,
    ],
)(_stats_body)


def _pq_body(sd_ref, ss_ref, p_ref, q_ref):
    sd = sd_ref[0] + sd_ref[1]
    ss = ss_ref[0] + ss_ref[1]
    p_ref[...] = lax.rsqrt(jnp.maximum(sd, jnp.float32(1e-30)))
    q_ref[...] = lax.rsqrt(jnp.maximum(ss, jnp.float32(1e-30)))


def _pq(sd3, ss3):
    return pl.pallas_call(
        _pq_body,
        out_shape=[
            jax.ShapeDtypeStruct((NP // 128, 128), jnp.float32),
            jax.ShapeDtypeStruct((NP // 128, 128), jnp.float32),
        ],
    )(sd3, ss3)


def _scale_body(f_ref, s_ref, o_ref):
    o_ref[...] = f_ref[...] * s_ref[...]


def _scale_rows(feat, col):
    bp = 1024
    return pl.pallas_call(
        _scale_body,
        grid=(NP // bp,),
        in_specs=[
            pl.BlockSpec((bp, 128), lambda i: (i, 0)),
            pl.BlockSpec((bp, 1), lambda i: (i, 0)),
        ],
        out_specs=pl.BlockSpec((bp, 128), lambda i: (i, 0)),
        out_shape=jax.ShapeDtypeStruct((NP, 128), jnp.float32),
    )(feat, col)


def _merge_body(pt_ref, p_ref, q_ref, h_ref, g_ref):
    h = (pt_ref[0] + pt_ref[1]) * p_ref[...]
    h_ref[...] = h
    g_ref[...] = h * q_ref[...]


def _merge(part, p_col, q_col):
    bp = 1024
    return pl.pallas_call(
        _merge_body,
        grid=(NP // bp,),
        in_specs=[
            pl.BlockSpec((NC, bp, 128), lambda i: (0, i, 0)),
            pl.BlockSpec((bp, 1), lambda i: (i, 0)),
            pl.BlockSpec((bp, 1), lambda i: (i, 0)),
        ],
        out_specs=[
            pl.BlockSpec((bp, 128), lambda i: (i, 0)),
            pl.BlockSpec((bp, 128), lambda i: (i, 0)),
        ],
        out_shape=[
            jax.ShapeDtypeStruct((NP, 128), jnp.float32),
            jax.ShapeDtypeStruct((NP, 128), jnp.float32),
        ],
    )(part, p_col, q_col)


def _prop_body(g_hbm, w_hbm, srci_hbm, dsti_hbm, out_hbm,
               idx_s, idx_d, wv, rows, zbuf, acc_sh, sem_g):
    cid = lax.axis_index("c")
    sid = lax.axis_index("s")
    wid = cid * NS + sid

    def _z(i, _):
        for t in range(8):
            zbuf[i, pl.ds(t * 16, 16)] = jnp.zeros((16,), jnp.float32)
        return 0
    lax.fori_loop(0, 64, _z, 0)

    def _zc(i, _):
        pltpu.sync_copy(zbuf, acc_sh.at[pl.ds(sid * RPW + i * 64, 64), :])
        return 0
    lax.fori_loop(0, RPW // 64, _zc, 0)
    plsc.subcore_barrier()

    def _chunk(j, _):
        base = wid * EW + j * CE
        pltpu.sync_copy(srci_hbm.at[pl.ds(base, CE)], idx_s)
        pltpu.sync_copy(dsti_hbm.at[pl.ds(base, CE)], idx_d)
        pltpu.sync_copy(w_hbm.at[pl.ds(base, CE)], wv)
        pltpu.async_copy(g_hbm.at[idx_s], rows, sem_g).wait()

        def _mul(i, _):
            s16 = plsc.load_gather(wv, [jnp.full((16,), i, jnp.int32)])
            for t in range(8):
                sl = pl.ds(t * 16, 16)
                rows[i, sl] = rows[i, sl] * s16
            return 0
        lax.fori_loop(0, CE, _mul, 0)
        pltpu.sync_copy(rows, acc_sh.at[idx_d], add=True)
        return 0

    lax.fori_loop(0, NCH, _chunk, 0)
    plsc.subcore_barrier()
    sl = pl.ds(sid * RPW, RPW)
    pltpu.sync_copy(acc_sh.at[sl, :], out_hbm.at[cid, sl, :])


_prop_call = functools.partial(
    pl.kernel,
    out_type=jax.ShapeDtypeStruct((NC, NP, 128), jnp.float32),
    mesh=_mesh,
    scratch_types=[
        pltpu.VMEM((CE,), jnp.int32),
        pltpu.VMEM((CE,), jnp.int32),
        pltpu.VMEM((CE,), jnp.float32),
        pltpu.VMEM((CE, 128), jnp.float32),
        pltpu.VMEM((64, 128), jnp.float32),
        pltpu.VMEM_SHARED((NP, 128), jnp.float32),
        pltpu.SemaphoreType.DMA,
    ],
)(_prop_body)


def _final_body(h1_ref, h2_ref, h3_ref, fd_ref, c_ref, o_ref):
    cst = c_ref[...]
    hts = []
    for k, href in enumerate((h1_ref, h2_ref, h3_ref)):
        h = href[...]
        mean = jnp.mean(h, axis=1, keepdims=True)
        var = jnp.mean(jnp.square(h - mean), axis=1, keepdims=True) \
            + jnp.float32(1e-9)
        ht = (h - mean) * cst[k:k + 1, :] * lax.rsqrt(var) \
            + cst[3 + k:4 + k, :] + cst[6 + k:7 + k, :]
        hts.append(ht)
    hop_l = cst[9:10, :]
    hop_r = cst[10:11, :]
    a_l = jnp.sum(hts[0] * hop_l, axis=1, keepdims=True)
    ls = [jnp.sum(ht * hop_r, axis=1, keepdims=True) + a_l for ht in hts]
    ls = [jnp.where(l >= 0.0, l, l * jnp.float32(0.2)) for l in ls]
    m = jnp.maximum(jnp.maximum(ls[0], ls[1]), ls[2])
    ws = [jnp.exp(l - m) for l in ls]
    tot = ws[0] + ws[1] + ws[2]
    out = fd_ref[...]
    for ht, w in zip(hts, ws):
        out = out + ht * (w / tot)
    o_ref[...] = out


def _final(h1, h2, h3, fd, consts):
    bp = 1000
    return pl.pallas_call(
        _final_body,
        grid=(N // bp,),
        in_specs=[
            pl.BlockSpec((bp, 128), lambda i: (i, 0)),
            pl.BlockSpec((bp, 128), lambda i: (i, 0)),
            pl.BlockSpec((bp, 128), lambda i: (i, 0)),
            pl.BlockSpec((bp, 128), lambda i: (i, 0)),
            pl.BlockSpec((16, 128), lambda i: (0, 0)),
        ],
        out_specs=pl.BlockSpec((bp, 128), lambda i: (i, 0)),
        out_shape=jax.ShapeDtypeStruct((N, 128), jnp.float32),
    )(h1, h2, h3, fd, consts)


def kernel(x, edge_index, W_src, W_dst, b_dst, W_attn_src, W_attn_dst,
           scale, offset, hop_attn_l, hop_attn_r, position_emb):
    srci = edge_index[0]
    dsti = edge_index[1]
    wa_pad = jnp.concatenate(
        [W_attn_src, W_attn_dst, jnp.zeros((126, 128), jnp.float32)], axis=0)
    b_row = b_dst.reshape(1, 128)

    feat_src, feat_dst, attn, cmax = _proj(x, W_src, W_dst, wa_pad, b_row)
    asrc = attn[:, 0]
    adst = attn[:, 1]
    c_off = cmax[0, 0] + cmax[0, 1]
    c_vec = jnp.full((16,), c_off, jnp.float32)

    ex, sd2, ss2 = _stats_call(asrc, adst, srci, dsti, c_vec)

    p2, q2 = _pq(sd2.reshape(NC, NP // 128, 128),
                 ss2.reshape(NC, NP // 128, 128))
    p_col = p2.reshape(NP, 1)
    q_col = q2.reshape(NP, 1)

    feat0 = jnp.pad(feat_src, ((0, NP - N), (0, 0)))
    g = _scale_rows(feat0, q_col)

    hs = []
    for _ in range(K):
        part = _prop_call(g, ex, srci, dsti)
        h, g = _merge(part, p_col, q_col)
        hs.append(h)

    consts = jnp.concatenate([
        scale[:3, 0, 0, :],
        offset[:3, 0, 0, :],
        position_emb[:, 0, :],
        hop_attn_l.reshape(1, 128),
        hop_attn_r.reshape(1, 128),
        jnp.zeros((5, 128), jnp.float32),
    ], axis=0)

    rst = _final(hs[0][:N], hs[1][:N], hs[2][:N], feat_dst, consts)
    return rst.reshape(N, 1, D)


# speedup vs baseline: 11.5944x; 11.5944x over previous
import functools

import jax
import jax.numpy as jnp
from jax import lax
from jax.experimental import pallas as pl
from jax.experimental.pallas import tpu as pltpu
from jax.experimental.pallas import tpu_sc as plsc

N = 10000
E = 320000
D = 128
K = 3
NP = 10240
NC = 2
NS = 16
NW = NC * NS
EW = E // NW
CE = 80
NCH = EW // CE
RPW = NP // NS

_mesh = plsc.VectorSubcoreMesh(core_axis_name="c", subcore_axis_name="s")


def _proj_body(x_ref, ws_ref, wd_ref, wa_ref, b_ref, fs_ref, fd_ref, at_ref,
               cm_ref):
    xb = x_ref[...]
    dn = (((1,), (1,)), ((), ()))
    fs_ref[...] = lax.dot_general(xb, ws_ref[...], dn,
                                  preferred_element_type=jnp.float32)
    fd_ref[...] = lax.dot_general(xb, wd_ref[...], dn,
                                  preferred_element_type=jnp.float32) + b_ref[...]
    at = lax.dot_general(xb, wa_ref[...], dn,
                         preferred_element_type=jnp.float32)
    at_ref[...] = at
    m8 = jnp.broadcast_to(jnp.max(at, axis=0, keepdims=True), (8, 128))

    @pl.when(pl.program_id(0) == 0)
    def _():
        cm_ref[...] = m8

    @pl.when(pl.program_id(0) > 0)
    def _():
        cm_ref[...] = jnp.maximum(cm_ref[...], m8)


def _proj(x, w_src, w_dst, wa_pad, b_row):
    bp = 1000
    return pl.pallas_call(
        _proj_body,
        grid=(N // bp,),
        in_specs=[
            pl.BlockSpec((bp, 128), lambda i: (i, 0)),
            pl.BlockSpec((128, 128), lambda i: (0, 0)),
            pl.BlockSpec((128, 128), lambda i: (0, 0)),
            pl.BlockSpec((128, 128), lambda i: (0, 0)),
            pl.BlockSpec((1, 128), lambda i: (0, 0)),
        ],
        out_specs=[
            pl.BlockSpec((bp, 128), lambda i: (i, 0)),
            pl.BlockSpec((bp, 128), lambda i: (i, 0)),
            pl.BlockSpec((bp, 128), lambda i: (i, 0)),
            pl.BlockSpec((8, 128), lambda i: (0, 0)),
        ],
        out_shape=[
            jax.ShapeDtypeStruct((N, 128), jnp.float32),
            jax.ShapeDtypeStruct((N, 128), jnp.float32),
            jax.ShapeDtypeStruct((N, 128), jnp.float32),
            jax.ShapeDtypeStruct((8, 128), jnp.float32),
        ],
    )(x, w_src, w_dst, wa_pad, b_row)


def _stats_body(asrc_hbm, adst_hbm, srci_hbm, dsti_hbm, cvec_hbm,
                ex_hbm, sd_hbm, ss_hbm,
                idx_s, idx_d, vas, vad, vex, cv_v, szero, sd_sh, ss_sh,
                sem_a, sem_b):
    cid = lax.axis_index("c")
    sid = lax.axis_index("s")
    wid = cid * NS + sid

    pltpu.sync_copy(cvec_hbm, cv_v)
    cv = cv_v[...]

    def _z(i, _):
        szero[pl.ds(i * 16, 16)] = jnp.zeros((16,), jnp.float32)
        return 0
    lax.fori_loop(0, RPW // 16, _z, 0)
    pltpu.sync_copy(szero, sd_sh.at[pl.ds(sid * RPW, RPW)])
    pltpu.sync_copy(szero, ss_sh.at[pl.ds(sid * RPW, RPW)])
    plsc.subcore_barrier()

    def _chunk(j, _):
        base = wid * EW + j * CE
        pltpu.sync_copy(srci_hbm.at[pl.ds(base, CE)], idx_s)
        pltpu.sync_copy(dsti_hbm.at[pl.ds(base, CE)], idx_d)
        c1 = pltpu.async_copy(asrc_hbm.at[idx_s], vas, sem_a)
        c2 = pltpu.async_copy(adst_hbm.at[idx_d], vad, sem_b)
        c1.wait()
        c2.wait()
        for t in range(CE // 16):
            sl = pl.ds(t * 16, 16)
            v = vas[sl] + vad[sl]
            e = jnp.where(v >= 0.0, v, v * jnp.float32(0.2))
            vex[sl] = jnp.exp(e - cv)
        pltpu.sync_copy(vex, ex_hbm.at[pl.ds(base, CE)])
        pltpu.sync_copy(vex, sd_sh.at[idx_d], add=True)
        pltpu.sync_copy(vex, ss_sh.at[idx_s], add=True)
        return 0

    lax.fori_loop(0, NCH, _chunk, 0)
    plsc.subcore_barrier()
    sl = pl.ds(sid * RPW, RPW)
    pltpu.sync_copy(sd_sh.at[sl], sd_hbm.at[cid, sl])
    pltpu.sync_copy(ss_sh.at[sl], ss_hbm.at[cid, sl])


_stats_call = functools.partial(
    pl.kernel,
    out_type=(
        jax.ShapeDtypeStruct((E,), jnp.float32),
        jax.ShapeDtypeStruct((NC, NP), jnp.float32),
        jax.ShapeDtypeStruct((NC, NP), jnp.float32),
    ),
    mesh=_mesh,
    scratch_types=[
        pltpu.VMEM((CE,), jnp.int32),
        pltpu.VMEM((CE,), jnp.int32),
        pltpu.VMEM((CE,), jnp.float32),
        pltpu.VMEM((CE,), jnp.float32),
        pltpu.VMEM((CE,), jnp.float32),
        pltpu.VMEM((16,), jnp.float32),
        pltpu.VMEM((RPW,), jnp.float32),
        pltpu.VMEM_SHARED((NP,), jnp.float32),
        pltpu.VMEM_SHARED((NP,), jnp.float32),
        pltpu.SemaphoreType.DMA,
        pltpu.SemaphoreType.DMA,
    ],
)(_stats_body)


def _pq_body(sd_ref, ss_ref, p_ref, q_ref):
    sd = sd_ref[0] + sd_ref[1]
    ss = ss_ref[0] + ss_ref[1]
    p_ref[...] = lax.rsqrt(jnp.maximum(sd, jnp.float32(1e-30)))
    q_ref[...] = lax.rsqrt(jnp.maximum(ss, jnp.float32(1e-30)))


def _pq(sd3, ss3):
    return pl.pallas_call(
        _pq_body,
        out_shape=[
            jax.ShapeDtypeStruct((NP // 128, 128), jnp.float32),
            jax.ShapeDtypeStruct((NP // 128, 128), jnp.float32),
        ],
    )(sd3, ss3)


def _scale_body(f_ref, s_ref, o_ref):
    o_ref[...] = f_ref[...] * s_ref[...]


def _scale_rows(feat, col):
    bp = 1024
    return pl.pallas_call(
        _scale_body,
        grid=(NP // bp,),
        in_specs=[
            pl.BlockSpec((bp, 128), lambda i: (i, 0)),
            pl.BlockSpec((bp, 1), lambda i: (i, 0)),
        ],
        out_specs=pl.BlockSpec((bp, 128), lambda i: (i, 0)),
        out_shape=jax.ShapeDtypeStruct((NP, 128), jnp.float32),
    )(feat, col)


def _merge_body(pt_ref, p_ref, q_ref, h_ref, g_ref):
    h = (pt_ref[0] + pt_ref[1]) * p_ref[...]
    h_ref[...] = h
    g_ref[...] = h * q_ref[...]


def _merge(part, p_col, q_col):
    bp = 1024
    return pl.pallas_call(
        _merge_body,
        grid=(NP // bp,),
        in_specs=[
            pl.BlockSpec((NC, bp, 128), lambda i: (0, i, 0)),
            pl.BlockSpec((bp, 1), lambda i: (i, 0)),
            pl.BlockSpec((bp, 1), lambda i: (i, 0)),
        ],
        out_specs=[
            pl.BlockSpec((bp, 128), lambda i: (i, 0)),
            pl.BlockSpec((bp, 128), lambda i: (i, 0)),
        ],
        out_shape=[
            jax.ShapeDtypeStruct((NP, 128), jnp.float32),
            jax.ShapeDtypeStruct((NP, 128), jnp.float32),
        ],
    )(part, p_col, q_col)


def _prop_body(g_hbm, w_hbm, srci_hbm, dsti_hbm, out_hbm,
               idx_s, idx_d, wv, rows, zbuf, acc_sh, sem_g):
    cid = lax.axis_index("c")
    sid = lax.axis_index("s")
    wid = cid * NS + sid

    def _z(i, _):
        for t in range(8):
            zbuf[i, pl.ds(t * 16, 16)] = jnp.zeros((16,), jnp.float32)
        return 0
    lax.fori_loop(0, 64, _z, 0)

    def _zc(i, _):
        pltpu.sync_copy(zbuf, acc_sh.at[pl.ds(sid * RPW + i * 64, 64), :])
        return 0
    lax.fori_loop(0, RPW // 64, _zc, 0)
    plsc.subcore_barrier()

    def _chunk(j, _):
        base = wid * EW + j * CE
        pltpu.sync_copy(srci_hbm.at[pl.ds(base, CE)], idx_s)
        pltpu.sync_copy(dsti_hbm.at[pl.ds(base, CE)], idx_d)
        pltpu.sync_copy(w_hbm.at[pl.ds(base, CE)], wv)
        pltpu.async_copy(g_hbm.at[idx_s], rows, sem_g).wait()

        def _mul(g, _):
            w16 = wv[pl.ds(g * 16, 16)]
            for u in range(16):
                i = g * 16 + u
                s = w16[u]
                for t in range(8):
                    sl = pl.ds(t * 16, 16)
                    rows[i, sl] = rows[i, sl] * s
            return 0
        lax.fori_loop(0, CE // 16, _mul, 0)
        pltpu.sync_copy(rows, acc_sh.at[idx_d], add=True)
        return 0

    lax.fori_loop(0, NCH, _chunk, 0)
    plsc.subcore_barrier()
    sl = pl.ds(sid * RPW, RPW)
    pltpu.sync_copy(acc_sh.at[sl, :], out_hbm.at[cid, sl, :])


_prop_call = functools.partial(
    pl.kernel,
    out_type=jax.ShapeDtypeStruct((NC, NP, 128), jnp.float32),
    mesh=_mesh,
    scratch_types=[
        pltpu.VMEM((CE,), jnp.int32),
        pltpu.VMEM((CE,), jnp.int32),
        pltpu.VMEM((CE,), jnp.float32),
        pltpu.VMEM((CE, 128), jnp.float32),
        pltpu.VMEM((64, 128), jnp.float32),
        pltpu.VMEM_SHARED((NP, 128), jnp.float32),
        pltpu.SemaphoreType.DMA,
    ],
)(_prop_body)


def _final_body(h1_ref, h2_ref, h3_ref, fd_ref, c_ref, o_ref):
    cst = c_ref[...]
    hts = []
    for k, href in enumerate((h1_ref, h2_ref, h3_ref)):
        h = href[...]
        mean = jnp.mean(h, axis=1, keepdims=True)
        var = jnp.mean(jnp.square(h - mean), axis=1, keepdims=True) \
            + jnp.float32(1e-9)
        ht = (h - mean) * cst[k:k + 1, :] * lax.rsqrt(var) \
            + cst[3 + k:4 + k, :] + cst[6 + k:7 + k, :]
        hts.append(ht)
    hop_l = cst[9:10, :]
    hop_r = cst[10:11, :]
    a_l = jnp.sum(hts[0] * hop_l, axis=1, keepdims=True)
    ls = [jnp.sum(ht * hop_r, axis=1, keepdims=True) + a_l for ht in hts]
    ls = [jnp.where(l >= 0.0, l, l * jnp.float32(0.2)) for l in ls]
    m = jnp.maximum(jnp.maximum(ls[0], ls[1]), ls[2])
    ws = [jnp.exp(l - m) for l in ls]
    tot = ws[0] + ws[1] + ws[2]
    out = fd_ref[...]
    for ht, w in zip(hts, ws):
        out = out + ht * (w / tot)
    o_ref[...] = out


def _final(h1, h2, h3, fd, consts):
    bp = 1000
    return pl.pallas_call(
        _final_body,
        grid=(N // bp,),
        in_specs=[
            pl.BlockSpec((bp, 128), lambda i: (i, 0)),
            pl.BlockSpec((bp, 128), lambda i: (i, 0)),
            pl.BlockSpec((bp, 128), lambda i: (i, 0)),
            pl.BlockSpec((bp, 128), lambda i: (i, 0)),
            pl.BlockSpec((16, 128), lambda i: (0, 0)),
        ],
        out_specs=pl.BlockSpec((bp, 128), lambda i: (i, 0)),
        out_shape=jax.ShapeDtypeStruct((N, 128), jnp.float32),
    )(h1, h2, h3, fd, consts)


def kernel(x, edge_index, W_src, W_dst, b_dst, W_attn_src, W_attn_dst,
           scale, offset, hop_attn_l, hop_attn_r, position_emb):
    srci = edge_index[0]
    dsti = edge_index[1]
    wa_pad = jnp.concatenate(
        [W_attn_src, W_attn_dst, jnp.zeros((126, 128), jnp.float32)], axis=0)
    b_row = b_dst.reshape(1, 128)

    feat_src, feat_dst, attn, cmax = _proj(x, W_src, W_dst, wa_pad, b_row)
    asrc = attn[:, 0]
    adst = attn[:, 1]
    c_off = cmax[0, 0] + cmax[0, 1]
    c_vec = jnp.full((16,), c_off, jnp.float32)

    ex, sd2, ss2 = _stats_call(asrc, adst, srci, dsti, c_vec)

    p2, q2 = _pq(sd2.reshape(NC, NP // 128, 128),
                 ss2.reshape(NC, NP // 128, 128))
    p_col = p2.reshape(NP, 1)
    q_col = q2.reshape(NP, 1)

    feat0 = jnp.pad(feat_src, ((0, NP - N), (0, 0)))
    g = _scale_rows(feat0, q_col)

    hs = []
    for _ in range(K):
        part = _prop_call(g, ex, srci, dsti)
        h, g = _merge(part, p_col, q_col)
        hs.append(h)

    consts = jnp.concatenate([
        scale[:3, 0, 0, :],
        offset[:3, 0, 0, :],
        position_emb[:, 0, :],
        hop_attn_l.reshape(1, 128),
        hop_attn_r.reshape(1, 128),
        jnp.zeros((5, 128), jnp.float32),
    ], axis=0)

    rst = _final(hs[0][:N], hs[1][:N], hs[2][:N], feat_dst, consts)
    return rst.reshape(N, 1, D)


# speedup vs baseline: 17.7126x; 1.5277x over previous
import functools

import jax
import jax.numpy as jnp
from jax import lax
from jax.experimental import pallas as pl
from jax.experimental.pallas import tpu as pltpu
from jax.experimental.pallas import tpu_sc as plsc

N = 10000
E = 320000
D = 128
K = 3
NP = 10240
NC = 2
NS = 16
NW = NC * NS
EW = E // NW
CE = 80
NCH = EW // CE
GRP = 5
NGRP = NCH // GRP
CEP = 16
NCHP = EW // CEP
NGRPP = NCHP // GRP
RPW = NP // NS

_mesh = plsc.VectorSubcoreMesh(core_axis_name="c", subcore_axis_name="s")


def _proj_body(x_ref, ws_ref, wd_ref, wa_ref, b_ref, fs_ref, fd_ref, at_ref,
               cm_ref):
    xb = x_ref[...]
    dn = (((1,), (1,)), ((), ()))
    fs_ref[...] = lax.dot_general(xb, ws_ref[...], dn,
                                  preferred_element_type=jnp.float32)
    fd_ref[...] = lax.dot_general(xb, wd_ref[...], dn,
                                  preferred_element_type=jnp.float32) + b_ref[...]
    at = lax.dot_general(xb, wa_ref[...], dn,
                         preferred_element_type=jnp.float32)
    at_ref[...] = at
    m8 = jnp.broadcast_to(jnp.max(at, axis=0, keepdims=True), (8, 128))

    @pl.when(pl.program_id(0) == 0)
    def _():
        cm_ref[...] = m8

    @pl.when(pl.program_id(0) > 0)
    def _():
        cm_ref[...] = jnp.maximum(cm_ref[...], m8)


def _proj(x, w_src, w_dst, wa_pad, b_row):
    bp = 1000
    return pl.pallas_call(
        _proj_body,
        grid=(N // bp,),
        in_specs=[
            pl.BlockSpec((bp, 128), lambda i: (i, 0)),
            pl.BlockSpec((128, 128), lambda i: (0, 0)),
            pl.BlockSpec((128, 128), lambda i: (0, 0)),
            pl.BlockSpec((128, 128), lambda i: (0, 0)),
            pl.BlockSpec((1, 128), lambda i: (0, 0)),
        ],
        out_specs=[
            pl.BlockSpec((bp, 128), lambda i: (i, 0)),
            pl.BlockSpec((bp, 128), lambda i: (i, 0)),
            pl.BlockSpec((bp, 128), lambda i: (i, 0)),
            pl.BlockSpec((8, 128), lambda i: (0, 0)),
        ],
        out_shape=[
            jax.ShapeDtypeStruct((N, 128), jnp.float32),
            jax.ShapeDtypeStruct((N, 128), jnp.float32),
            jax.ShapeDtypeStruct((N, 128), jnp.float32),
            jax.ShapeDtypeStruct((8, 128), jnp.float32),
        ],
    )(x, w_src, w_dst, wa_pad, b_row)


def _stats_body(asrc_hbm, adst_hbm, srci_hbm, dsti_hbm, cvec_hbm,
                ex_hbm, sd_hbm, ss_hbm,
                sall, dall, isrc, dsc, vas, vad, vex, cv_v, szero,
                sd_sh, ss_sh, semi, semg, semsc):
    cid = lax.axis_index("c")
    sid = lax.axis_index("s")
    wid = cid * NS + sid
    ebase = wid * EW

    pltpu.sync_copy(cvec_hbm, cv_v)
    cv = cv_v[...]

    l1 = pltpu.async_copy(srci_hbm.at[pl.ds(ebase, EW)], sall, semi)
    l2 = pltpu.async_copy(dsti_hbm.at[pl.ds(ebase, EW)], dall, semg[0])

    def _z(i, _):
        szero[pl.ds(i * 16, 16)] = jnp.zeros((16,), jnp.float32)
        return 0
    lax.fori_loop(0, RPW // 16, _z, 0)
    pltpu.sync_copy(szero, sd_sh.at[pl.ds(sid * RPW, RPW)])
    pltpu.sync_copy(szero, ss_sh.at[pl.ds(sid * RPW, RPW)])
    plsc.subcore_barrier()
    l1.wait()
    l2.wait()

    def _chunk(j, _):
        cb = j * CE
        for g in range(CE // 16):
            so = pl.ds(g * 16, 16)
            bo = pl.ds(cb + g * 16, 16)
            isrc[0][so] = sall[bo]
            dsc[0][so] = dall[bo]
        d1 = pltpu.async_copy(asrc_hbm.at[isrc[0]], vas[0], semg[0])
        d2 = pltpu.async_copy(adst_hbm.at[dsc[0]], vad[0], semg[1])
        d1.wait()
        d2.wait()
        for g in range(CE // 16):
            so = pl.ds(g * 16, 16)
            v = vas[0][so] + vad[0][so]
            e = jnp.where(v >= 0.0, v, v * jnp.float32(0.2))
            vex[0][so] = jnp.exp(e - cv)
        pltpu.sync_copy(vex[0], ex_hbm.at[pl.ds(ebase + cb, CE)])
        pltpu.sync_copy(vex[0], sd_sh.at[dsc[0]], add=True)
        pltpu.sync_copy(vex[0], ss_sh.at[isrc[0]], add=True)
        return 0

    lax.fori_loop(0, NCH, _chunk, 0)
    plsc.subcore_barrier()
    sl = pl.ds(sid * RPW, RPW)
    pltpu.sync_copy(sd_sh.at[sl], sd_hbm.at[cid, sl])
    pltpu.sync_copy(ss_sh.at[sl], ss_hbm.at[cid, sl])


_stats_call = functools.partial(
    pl.kernel,
    out_type=(
        jax.ShapeDtypeStruct((E,), jnp.float32),
        jax.ShapeDtypeStruct((NC, NP), jnp.float32),
        jax.ShapeDtypeStruct((NC, NP), jnp.float32),
    ),
    mesh=_mesh,
    scratch_types=(
        pltpu.VMEM((EW,), jnp.int32),
        pltpu.VMEM((EW,), jnp.int32),
        [pltpu.VMEM((CE,), jnp.int32)] * 1,
        [pltpu.VMEM((CE,), jnp.int32)] * 1,
        [pltpu.VMEM((CE,), jnp.float32)] * 1,
        [pltpu.VMEM((CE,), jnp.float32)] * 1,
        [pltpu.VMEM((CE,), jnp.float32)] * 1,
        pltpu.VMEM((16,), jnp.float32),
        pltpu.VMEM((RPW,), jnp.float32),
        pltpu.VMEM_SHARED((NP,), jnp.float32),
        pltpu.VMEM_SHARED((NP,), jnp.float32),
        pltpu.SemaphoreType.DMA,
        [pltpu.SemaphoreType.DMA] * 2,
        pltpu.SemaphoreType.DMA,
    ),
)(_stats_body)


def _pq_body(sd_ref, ss_ref, p_ref, q_ref):
    sd = sd_ref[0] + sd_ref[1]
    ss = ss_ref[0] + ss_ref[1]
    p_ref[...] = lax.rsqrt(jnp.maximum(sd, jnp.float32(1e-30)))
    q_ref[...] = lax.rsqrt(jnp.maximum(ss, jnp.float32(1e-30)))


def _pq(sd3, ss3):
    return pl.pallas_call(
        _pq_body,
        out_shape=[
            jax.ShapeDtypeStruct((NP // 128, 128), jnp.float32),
            jax.ShapeDtypeStruct((NP // 128, 128), jnp.float32),
        ],
    )(sd3, ss3)


def _scale_body(f_ref, s_ref, o_ref):
    o_ref[...] = f_ref[...] * s_ref[...]


def _scale_rows(feat, col):
    bp = 1024
    return pl.pallas_call(
        _scale_body,
        grid=(NP // bp,),
        in_specs=[
            pl.BlockSpec((bp, 128), lambda i: (i, 0)),
            pl.BlockSpec((bp, 1), lambda i: (i, 0)),
        ],
        out_specs=pl.BlockSpec((bp, 128), lambda i: (i, 0)),
        out_shape=jax.ShapeDtypeStruct((NP, 128), jnp.float32),
    )(feat, col)


def _merge_body(pt_ref, p_ref, q_ref, h_ref, g_ref):
    h = (pt_ref[0] + pt_ref[1]) * p_ref[...]
    h_ref[...] = h
    g_ref[...] = h * q_ref[...]


def _merge(part, p_col, q_col):
    bp = 1024
    return pl.pallas_call(
        _merge_body,
        grid=(NP // bp,),
        in_specs=[
            pl.BlockSpec((NC, bp, 128), lambda i: (0, i, 0)),
            pl.BlockSpec((bp, 1), lambda i: (i, 0)),
            pl.BlockSpec((bp, 1), lambda i: (i, 0)),
        ],
        out_specs=[
            pl.BlockSpec((bp, 128), lambda i: (i, 0)),
            pl.BlockSpec((bp, 128), lambda i: (i, 0)),
        ],
        out_shape=[
            jax.ShapeDtypeStruct((NP, 128), jnp.float32),
            jax.ShapeDtypeStruct((NP, 128), jnp.float32),
        ],
    )(part, p_col, q_col)


def _prop_body(g_hbm, w_hbm, srci_hbm, dsti_hbm, out_hbm,
               sall, dall, wall, isrc, dsc, rows, zbuf, acc_sh,
               semi, semg, semsc):
    cid = lax.axis_index("c")
    sid = lax.axis_index("s")
    wid = cid * NS + sid
    ebase = wid * EW

    l1 = pltpu.async_copy(srci_hbm.at[pl.ds(ebase, EW)], sall, semi)
    l2 = pltpu.async_copy(dsti_hbm.at[pl.ds(ebase, EW)], dall, semg[0])
    l3 = pltpu.async_copy(w_hbm.at[pl.ds(ebase, EW)], wall, semg[1])

    def _z(i, _):
        for t in range(8):
            zbuf[i, pl.ds(t * 16, 16)] = jnp.zeros((16,), jnp.float32)
        return 0
    lax.fori_loop(0, 16, _z, 0)

    def _zc(i, _):
        pltpu.sync_copy(zbuf, acc_sh.at[pl.ds(sid * RPW + i * 16, 16), :])
        return 0
    lax.fori_loop(0, RPW // 16, _zc, 0)
    plsc.subcore_barrier()
    l1.wait()
    l2.wait()
    l3.wait()

    def _chunk(j, _):
        cb = j * CE
        for g in range(CE // 16):
            so = pl.ds(g * 16, 16)
            bo = pl.ds(cb + g * 16, 16)
            isrc[0][so] = sall[bo]
            dsc[0][so] = dall[bo]
        pltpu.async_copy(g_hbm.at[isrc[0]], rows[0], semg[0]).wait()

        def _mul(g, _):
            w16 = wall[pl.ds(cb + g * 16, 16)]
            for u in range(16):
                s = w16[u]
                for t in range(8):
                    sl = pl.ds(t * 16, 16)
                    rows[0][g * 16 + u, sl] = rows[0][g * 16 + u, sl] * s
            return 0
        lax.fori_loop(0, CE // 16, _mul, 0)
        pltpu.sync_copy(rows[0], acc_sh.at[dsc[0]], add=True)
        return 0

    lax.fori_loop(0, NCH, _chunk, 0)
    plsc.subcore_barrier()
    sl = pl.ds(sid * RPW, RPW)
    pltpu.sync_copy(acc_sh.at[sl, :], out_hbm.at[cid, sl, :])


_prop_call = functools.partial(
    pl.kernel,
    out_type=jax.ShapeDtypeStruct((NC, NP, 128), jnp.float32),
    mesh=_mesh,
    scratch_types=(
        pltpu.VMEM((EW,), jnp.int32),
        pltpu.VMEM((EW,), jnp.int32),
        pltpu.VMEM((EW,), jnp.float32),
        [pltpu.VMEM((CE,), jnp.int32)] * 1,
        [pltpu.VMEM((CE,), jnp.int32)] * 1,
        [pltpu.VMEM((CE, 128), jnp.float32)] * 1,
        pltpu.VMEM((16, 128), jnp.float32),
        pltpu.VMEM_SHARED((NP, 128), jnp.float32),
        pltpu.SemaphoreType.DMA,
        [pltpu.SemaphoreType.DMA] * 2,
        pltpu.SemaphoreType.DMA,
    ),
)(_prop_body)


def _final_body(h1_ref, h2_ref, h3_ref, fd_ref, c_ref, o_ref):
    cst = c_ref[...]
    hts = []
    for k, href in enumerate((h1_ref, h2_ref, h3_ref)):
        h = href[...]
        mean = jnp.mean(h, axis=1, keepdims=True)
        var = jnp.mean(jnp.square(h - mean), axis=1, keepdims=True) \
            + jnp.float32(1e-9)
        ht = (h - mean) * cst[k:k + 1, :] * lax.rsqrt(var) \
            + cst[3 + k:4 + k, :] + cst[6 + k:7 + k, :]
        hts.append(ht)
    hop_l = cst[9:10, :]
    hop_r = cst[10:11, :]
    a_l = jnp.sum(hts[0] * hop_l, axis=1, keepdims=True)
    ls = [jnp.sum(ht * hop_r, axis=1, keepdims=True) + a_l for ht in hts]
    ls = [jnp.where(l >= 0.0, l, l * jnp.float32(0.2)) for l in ls]
    m = jnp.maximum(jnp.maximum(ls[0], ls[1]), ls[2])
    ws = [jnp.exp(l - m) for l in ls]
    tot = ws[0] + ws[1] + ws[2]
    out = fd_ref[...]
    for ht, w in zip(hts, ws):
        out = out + ht * (w / tot)
    o_ref[...] = out


def _final(h1, h2, h3, fd, consts):
    bp = 1000
    return pl.pallas_call(
        _final_body,
        grid=(N // bp,),
        in_specs=[
            pl.BlockSpec((bp, 128), lambda i: (i, 0)),
            pl.BlockSpec((bp, 128), lambda i: (i, 0)),
            pl.BlockSpec((bp, 128), lambda i: (i, 0)),
            pl.BlockSpec((bp, 128), lambda i: (i, 0)),
            pl.BlockSpec((16, 128), lambda i: (0, 0)),
        ],
        out_specs=pl.BlockSpec((bp, 128), lambda i: (i, 0)),
        out_shape=jax.ShapeDtypeStruct((N, 128), jnp.float32),
    )(h1, h2, h3, fd, consts)


def kernel(x, edge_index, W_src, W_dst, b_dst, W_attn_src, W_attn_dst,
           scale, offset, hop_attn_l, hop_attn_r, position_emb):
    srci = edge_index[0]
    dsti = edge_index[1]
    wa_pad = jnp.concatenate(
        [W_attn_src, W_attn_dst, jnp.zeros((126, 128), jnp.float32)], axis=0)
    b_row = b_dst.reshape(1, 128)

    feat_src, feat_dst, attn, cmax = _proj(x, W_src, W_dst, wa_pad, b_row)
    asrc = attn[:, 0]
    adst = attn[:, 1]
    c_off = cmax[0, 0] + cmax[0, 1]
    c_vec = jnp.full((16,), c_off, jnp.float32)

    ex, sd2, ss2 = _stats_call(asrc, adst, srci, dsti, c_vec)

    p2, q2 = _pq(sd2.reshape(NC, NP // 128, 128),
                 ss2.reshape(NC, NP // 128, 128))
    p_col = p2.reshape(NP, 1)
    q_col = q2.reshape(NP, 1)

    feat0 = jnp.pad(feat_src, ((0, NP - N), (0, 0)))
    g = _scale_rows(feat0, q_col)

    hs = []
    for _ in range(K):
        part = _prop_call(g, ex, srci, dsti)
        h, g = _merge(part, p_col, q_col)
        hs.append(h)

    consts = jnp.concatenate([
        scale[:3, 0, 0, :],
        offset[:3, 0, 0, :],
        position_emb[:, 0, :],
        hop_attn_l.reshape(1, 128),
        hop_attn_r.reshape(1, 128),
        jnp.zeros((5, 128), jnp.float32),
    ], axis=0)

    rst = _final(hs[0][:N], hs[1][:N], hs[2][:N], feat_dst, consts)
    return rst.reshape(N, 1, D)


# speedup vs baseline: 20.3811x; 1.1507x over previous
import functools

import jax
import jax.numpy as jnp
from jax import lax
from jax.experimental import pallas as pl
from jax.experimental.pallas import tpu as pltpu
from jax.experimental.pallas import tpu_sc as plsc

N = 10000
E = 320000
D = 128
K = 3
NP = 10240
NC = 2
NS = 16
NW = NC * NS
EW = E // NW
CE = 80
NCH = EW // CE
GRP = 5
NGRP = NCH // GRP
CEP = 16
NCHP = EW // CEP
NGRPP = NCHP // GRP
RPW = NP // NS

_mesh = plsc.VectorSubcoreMesh(core_axis_name="c", subcore_axis_name="s")


def _proj_body(x_ref, ws_ref, wd_ref, wa_ref, b_ref, fs_ref, fd_ref, at_ref,
               cm_ref):
    xb = x_ref[...]
    dn = (((1,), (1,)), ((), ()))
    fs_ref[...] = lax.dot_general(xb, ws_ref[...], dn,
                                  preferred_element_type=jnp.float32)
    fd_ref[...] = lax.dot_general(xb, wd_ref[...], dn,
                                  preferred_element_type=jnp.float32) + b_ref[...]
    at = lax.dot_general(xb, wa_ref[...], dn,
                         preferred_element_type=jnp.float32)
    at_ref[...] = at
    m8 = jnp.broadcast_to(jnp.max(at, axis=0, keepdims=True), (8, 128))

    @pl.when(pl.program_id(0) == 0)
    def _():
        cm_ref[...] = m8

    @pl.when(pl.program_id(0) > 0)
    def _():
        cm_ref[...] = jnp.maximum(cm_ref[...], m8)


def _proj(x, w_src, w_dst, wa_pad, b_row):
    bp = 1000
    return pl.pallas_call(
        _proj_body,
        grid=(N // bp,),
        in_specs=[
            pl.BlockSpec((bp, 128), lambda i: (i, 0)),
            pl.BlockSpec((128, 128), lambda i: (0, 0)),
            pl.BlockSpec((128, 128), lambda i: (0, 0)),
            pl.BlockSpec((128, 128), lambda i: (0, 0)),
            pl.BlockSpec((1, 128), lambda i: (0, 0)),
        ],
        out_specs=[
            pl.BlockSpec((bp, 128), lambda i: (i, 0)),
            pl.BlockSpec((bp, 128), lambda i: (i, 0)),
            pl.BlockSpec((bp, 128), lambda i: (i, 0)),
            pl.BlockSpec((8, 128), lambda i: (0, 0)),
        ],
        out_shape=[
            jax.ShapeDtypeStruct((N, 128), jnp.float32),
            jax.ShapeDtypeStruct((N, 128), jnp.float32),
            jax.ShapeDtypeStruct((N, 128), jnp.float32),
            jax.ShapeDtypeStruct((8, 128), jnp.float32),
        ],
    )(x, w_src, w_dst, wa_pad, b_row)


def _stats_body(asrc_hbm, adst_hbm, srci_hbm, dsti_hbm, cvec_hbm,
                ex_hbm, sd_hbm, ss_hbm,
                sall, dall, isrc, dsc, vas, vad, vex, cv_v, szero,
                sd_sh, ss_sh, semi, semg, semsc):
    cid = lax.axis_index("c")
    sid = lax.axis_index("s")
    wid = cid * NS + sid
    ebase = wid * EW

    pltpu.sync_copy(cvec_hbm, cv_v)
    cv = cv_v[...]

    l1 = pltpu.async_copy(srci_hbm.at[pl.ds(ebase, EW)], sall, semi)
    l2 = pltpu.async_copy(dsti_hbm.at[pl.ds(ebase, EW)], dall, semg[0])

    def _z(i, _):
        szero[pl.ds(i * 16, 16)] = jnp.zeros((16,), jnp.float32)
        return 0
    lax.fori_loop(0, RPW // 16, _z, 0)
    pltpu.sync_copy(szero, sd_sh.at[pl.ds(sid * RPW, RPW)])
    pltpu.sync_copy(szero, ss_sh.at[pl.ds(sid * RPW, RPW)])
    plsc.subcore_barrier()
    l1.wait()
    l2.wait()

    def _chunk(j, _):
        cb = j * CE
        for g in range(CE // 16):
            so = pl.ds(g * 16, 16)
            bo = pl.ds(cb + g * 16, 16)
            isrc[0][so] = sall[bo]
            dsc[0][so] = dall[bo]
        d1 = pltpu.async_copy(asrc_hbm.at[isrc[0]], vas[0], semg[0])
        d2 = pltpu.async_copy(adst_hbm.at[dsc[0]], vad[0], semg[1])
        d1.wait()
        d2.wait()
        for g in range(CE // 16):
            so = pl.ds(g * 16, 16)
            v = vas[0][so] + vad[0][so]
            e = jnp.where(v >= 0.0, v, v * jnp.float32(0.2))
            vex[0][so] = jnp.exp(e - cv)
        pltpu.sync_copy(vex[0], ex_hbm.at[pl.ds(ebase + cb, CE)])
        pltpu.sync_copy(vex[0], sd_sh.at[dsc[0]], add=True)
        pltpu.sync_copy(vex[0], ss_sh.at[isrc[0]], add=True)
        return 0

    lax.fori_loop(0, NCH, _chunk, 0)
    plsc.subcore_barrier()
    sl = pl.ds(sid * RPW, RPW)
    pltpu.sync_copy(sd_sh.at[sl], sd_hbm.at[cid, sl])
    pltpu.sync_copy(ss_sh.at[sl], ss_hbm.at[cid, sl])


_stats_call = functools.partial(
    pl.kernel,
    out_type=(
        jax.ShapeDtypeStruct((E,), jnp.float32),
        jax.ShapeDtypeStruct((NC, NP), jnp.float32),
        jax.ShapeDtypeStruct((NC, NP), jnp.float32),
    ),
    mesh=_mesh,
    scratch_types=(
        pltpu.VMEM((EW,), jnp.int32),
        pltpu.VMEM((EW,), jnp.int32),
        [pltpu.VMEM((CE,), jnp.int32)] * 1,
        [pltpu.VMEM((CE,), jnp.int32)] * 1,
        [pltpu.VMEM((CE,), jnp.float32)] * 1,
        [pltpu.VMEM((CE,), jnp.float32)] * 1,
        [pltpu.VMEM((CE,), jnp.float32)] * 1,
        pltpu.VMEM((16,), jnp.float32),
        pltpu.VMEM((RPW,), jnp.float32),
        pltpu.VMEM_SHARED((NP,), jnp.float32),
        pltpu.VMEM_SHARED((NP,), jnp.float32),
        pltpu.SemaphoreType.DMA,
        [pltpu.SemaphoreType.DMA] * 2,
        pltpu.SemaphoreType.DMA,
    ),
)(_stats_body)


def _pq_body(sd_ref, ss_ref, p_ref, q_ref):
    sd = sd_ref[0] + sd_ref[1]
    ss = ss_ref[0] + ss_ref[1]
    p_ref[...] = lax.rsqrt(jnp.maximum(sd, jnp.float32(1e-30)))
    q_ref[...] = lax.rsqrt(jnp.maximum(ss, jnp.float32(1e-30)))


def _pq(sd3, ss3):
    return pl.pallas_call(
        _pq_body,
        out_shape=[
            jax.ShapeDtypeStruct((NP // 128, 128), jnp.float32),
            jax.ShapeDtypeStruct((NP // 128, 128), jnp.float32),
        ],
    )(sd3, ss3)


def _scale_body(f_ref, s_ref, o_ref):
    o_ref[...] = f_ref[...] * s_ref[...]


def _scale_rows(feat, col):
    bp = 1024
    return pl.pallas_call(
        _scale_body,
        grid=(NP // bp,),
        in_specs=[
            pl.BlockSpec((bp, 128), lambda i: (i, 0)),
            pl.BlockSpec((bp, 1), lambda i: (i, 0)),
        ],
        out_specs=pl.BlockSpec((bp, 128), lambda i: (i, 0)),
        out_shape=jax.ShapeDtypeStruct((NP, 128), jnp.float32),
    )(feat, col)


def _merge_body(pt_ref, p_ref, q_ref, h_ref, g_ref):
    h = (pt_ref[0] + pt_ref[1]) * p_ref[...]
    h_ref[...] = h
    g_ref[...] = h * q_ref[...]


def _merge(part, p_col, q_col):
    bp = 1024
    return pl.pallas_call(
        _merge_body,
        grid=(NP // bp,),
        in_specs=[
            pl.BlockSpec((NC, bp, 128), lambda i: (0, i, 0)),
            pl.BlockSpec((bp, 1), lambda i: (i, 0)),
            pl.BlockSpec((bp, 1), lambda i: (i, 0)),
        ],
        out_specs=[
            pl.BlockSpec((bp, 128), lambda i: (i, 0)),
            pl.BlockSpec((bp, 128), lambda i: (i, 0)),
        ],
        out_shape=[
            jax.ShapeDtypeStruct((NP, 128), jnp.float32),
            jax.ShapeDtypeStruct((NP, 128), jnp.float32),
        ],
    )(part, p_col, q_col)


def _prop_body(g_hbm, w_hbm, srci_hbm, dsti_hbm, out_hbm,
               sall, dall, isrc, dsc, wv, rows, zbuf, acc_sh,
               semi, semg, semw, semsc):
    cid = lax.axis_index("c")
    sid = lax.axis_index("s")
    wid = cid * NS + sid
    ebase = wid * EW

    l1 = pltpu.async_copy(srci_hbm.at[pl.ds(ebase, EW)], sall, semi)
    l2 = pltpu.async_copy(dsti_hbm.at[pl.ds(ebase, EW)], dall, semg[0])

    def _z(i, _):
        for t in range(8):
            zbuf[i, pl.ds(t * 16, 16)] = jnp.zeros((16,), jnp.float32)
        return 0
    lax.fori_loop(0, 8, _z, 0)

    def _zc(i, _):
        pltpu.sync_copy(zbuf, acc_sh.at[pl.ds(sid * RPW + i * 8, 8), :])
        return 0
    lax.fori_loop(0, RPW // 8, _zc, 0)
    plsc.subcore_barrier()
    l1.wait()
    l2.wait()

    def _issue(j, b):
        cb = j * CE
        for g in range(CE // 16):
            so = pl.ds(g * 16, 16)
            bo = pl.ds(cb + g * 16, 16)
            isrc[b][so] = sall[bo]
            dsc[b][so] = dall[bo]
        dg = pltpu.async_copy(g_hbm.at[isrc[b]], rows[b], semg[b])
        dw = pltpu.async_copy(w_hbm.at[pl.ds(ebase + cb, CE)], wv[b],
                              semw[b])
        return dg, dw

    def _finish(b, dg, dw):
        dg.wait()
        dw.wait()

        def _mul(g, _):
            w16 = wv[b][pl.ds(g * 16, 16)]
            for u in range(16):
                s = w16[u]
                for t in range(8):
                    sl = pl.ds(t * 16, 16)
                    rows[b][g * 16 + u, sl] = rows[b][g * 16 + u, sl] * s
            return 0
        lax.fori_loop(0, CE // 16, _mul, 0)
        pltpu.sync_copy(rows[b], acc_sh.at[dsc[b]], add=True)

    def _pair(jj, _):
        d0 = _issue(jj * 2, 0)
        d1 = _issue(jj * 2 + 1, 1)
        _finish(0, *d0)
        _finish(1, *d1)
        return 0

    lax.fori_loop(0, NCH // 2, _pair, 0)
    dt = _issue(NCH - 1, 0)
    _finish(0, *dt)

    plsc.subcore_barrier()
    sl = pl.ds(sid * RPW, RPW)
    pltpu.sync_copy(acc_sh.at[sl, :], out_hbm.at[cid, sl, :])


_prop_call = functools.partial(
    pl.kernel,
    out_type=jax.ShapeDtypeStruct((NC, NP, 128), jnp.float32),
    mesh=_mesh,
    scratch_types=(
        pltpu.VMEM((EW,), jnp.int32),
        pltpu.VMEM((EW,), jnp.int32),
        [pltpu.VMEM((CE,), jnp.int32)] * 2,
        [pltpu.VMEM((CE,), jnp.int32)] * 2,
        [pltpu.VMEM((CE,), jnp.float32)] * 2,
        [pltpu.VMEM((CE, 128), jnp.float32)] * 2,
        pltpu.VMEM((8, 128), jnp.float32),
        pltpu.VMEM_SHARED((NP, 128), jnp.float32),
        pltpu.SemaphoreType.DMA,
        [pltpu.SemaphoreType.DMA] * 2,
        [pltpu.SemaphoreType.DMA] * 2,
        pltpu.SemaphoreType.DMA,
    ),
)(_prop_body)


def _final_body(h1_ref, h2_ref, h3_ref, fd_ref, c_ref, o_ref):
    cst = c_ref[...]
    hts = []
    for k, href in enumerate((h1_ref, h2_ref, h3_ref)):
        h = href[...]
        mean = jnp.mean(h, axis=1, keepdims=True)
        var = jnp.mean(jnp.square(h - mean), axis=1, keepdims=True) \
            + jnp.float32(1e-9)
        ht = (h - mean) * cst[k:k + 1, :] * lax.rsqrt(var) \
            + cst[3 + k:4 + k, :] + cst[6 + k:7 + k, :]
        hts.append(ht)
    hop_l = cst[9:10, :]
    hop_r = cst[10:11, :]
    a_l = jnp.sum(hts[0] * hop_l, axis=1, keepdims=True)
    ls = [jnp.sum(ht * hop_r, axis=1, keepdims=True) + a_l for ht in hts]
    ls = [jnp.where(l >= 0.0, l, l * jnp.float32(0.2)) for l in ls]
    m = jnp.maximum(jnp.maximum(ls[0], ls[1]), ls[2])
    ws = [jnp.exp(l - m) for l in ls]
    tot = ws[0] + ws[1] + ws[2]
    out = fd_ref[...]
    for ht, w in zip(hts, ws):
        out = out + ht * (w / tot)
    o_ref[...] = out


def _final(h1, h2, h3, fd, consts):
    bp = 1000
    return pl.pallas_call(
        _final_body,
        grid=(N // bp,),
        in_specs=[
            pl.BlockSpec((bp, 128), lambda i: (i, 0)),
            pl.BlockSpec((bp, 128), lambda i: (i, 0)),
            pl.BlockSpec((bp, 128), lambda i: (i, 0)),
            pl.BlockSpec((bp, 128), lambda i: (i, 0)),
            pl.BlockSpec((16, 128), lambda i: (0, 0)),
        ],
        out_specs=pl.BlockSpec((bp, 128), lambda i: (i, 0)),
        out_shape=jax.ShapeDtypeStruct((N, 128), jnp.float32),
    )(h1, h2, h3, fd, consts)


def kernel(x, edge_index, W_src, W_dst, b_dst, W_attn_src, W_attn_dst,
           scale, offset, hop_attn_l, hop_attn_r, position_emb):
    srci = edge_index[0]
    dsti = edge_index[1]
    wa_pad = jnp.concatenate(
        [W_attn_src, W_attn_dst, jnp.zeros((126, 128), jnp.float32)], axis=0)
    b_row = b_dst.reshape(1, 128)

    feat_src, feat_dst, attn, cmax = _proj(x, W_src, W_dst, wa_pad, b_row)
    asrc = attn[:, 0]
    adst = attn[:, 1]
    c_off = cmax[0, 0] + cmax[0, 1]
    c_vec = jnp.full((16,), c_off, jnp.float32)

    ex, sd2, ss2 = _stats_call(asrc, adst, srci, dsti, c_vec)

    p2, q2 = _pq(sd2.reshape(NC, NP // 128, 128),
                 ss2.reshape(NC, NP // 128, 128))
    p_col = p2.reshape(NP, 1)
    q_col = q2.reshape(NP, 1)

    feat0 = jnp.pad(feat_src, ((0, NP - N), (0, 0)))
    g = _scale_rows(feat0, q_col)

    hs = []
    for _ in range(K):
        part = _prop_call(g, ex, srci, dsti)
        h, g = _merge(part, p_col, q_col)
        hs.append(h)

    consts = jnp.concatenate([
        scale[:3, 0, 0, :],
        offset[:3, 0, 0, :],
        position_emb[:, 0, :],
        hop_attn_l.reshape(1, 128),
        hop_attn_r.reshape(1, 128),
        jnp.zeros((5, 128), jnp.float32),
    ], axis=0)

    rst = _final(hs[0][:N], hs[1][:N], hs[2][:N], feat_dst, consts)
    return rst.reshape(N, 1, D)


# speedup vs baseline: 21.9921x; 1.0790x over previous
import functools

import jax
import jax.numpy as jnp
from jax import lax
from jax.experimental import pallas as pl
from jax.experimental.pallas import tpu as pltpu
from jax.experimental.pallas import tpu_sc as plsc

N = 10000
E = 320000
D = 128
K = 3
NP = 10240
NC = 2
NS = 16
NW = NC * NS
EW = E // NW
CE = 80
NCH = EW // CE
GRP = 5
NGRP = NCH // GRP
CEP = 16
NCHP = EW // CEP
NGRPP = NCHP // GRP
RPW = NP // NS

_mesh = plsc.VectorSubcoreMesh(core_axis_name="c", subcore_axis_name="s")


def _proj_body(x_ref, ws_ref, wd_ref, wa_ref, b_ref, fs_ref, fd_ref, at_ref,
               cm_ref):
    xb = x_ref[...]
    dn = (((1,), (1,)), ((), ()))
    fs_ref[...] = lax.dot_general(xb, ws_ref[...], dn,
                                  preferred_element_type=jnp.float32)
    fd_ref[...] = lax.dot_general(xb, wd_ref[...], dn,
                                  preferred_element_type=jnp.float32) + b_ref[...]
    at = lax.dot_general(xb, wa_ref[...], dn,
                         preferred_element_type=jnp.float32)
    at_ref[...] = at
    m8 = jnp.broadcast_to(jnp.max(at, axis=0, keepdims=True), (8, 128))

    @pl.when(pl.program_id(0) == 0)
    def _():
        cm_ref[...] = m8

    @pl.when(pl.program_id(0) > 0)
    def _():
        cm_ref[...] = jnp.maximum(cm_ref[...], m8)


def _proj(x, w_src, w_dst, wa_pad, b_row):
    bp = 1000
    return pl.pallas_call(
        _proj_body,
        grid=(N // bp,),
        in_specs=[
            pl.BlockSpec((bp, 128), lambda i: (i, 0)),
            pl.BlockSpec((128, 128), lambda i: (0, 0)),
            pl.BlockSpec((128, 128), lambda i: (0, 0)),
            pl.BlockSpec((128, 128), lambda i: (0, 0)),
            pl.BlockSpec((1, 128), lambda i: (0, 0)),
        ],
        out_specs=[
            pl.BlockSpec((bp, 128), lambda i: (i, 0)),
            pl.BlockSpec((bp, 128), lambda i: (i, 0)),
            pl.BlockSpec((bp, 128), lambda i: (i, 0)),
            pl.BlockSpec((8, 128), lambda i: (0, 0)),
        ],
        out_shape=[
            jax.ShapeDtypeStruct((N, 128), jnp.float32),
            jax.ShapeDtypeStruct((N, 128), jnp.float32),
            jax.ShapeDtypeStruct((N, 128), jnp.float32),
            jax.ShapeDtypeStruct((8, 128), jnp.float32),
        ],
    )(x, w_src, w_dst, wa_pad, b_row)


def _stats_body(asrc_hbm, adst_hbm, srci_hbm, dsti_hbm, cvec_hbm,
                ex_hbm, sd_hbm, ss_hbm,
                sall, dall, isrc, dsc, vas, vad, vex, cv_v, szero,
                sd_sh, ss_sh, semi, semg, semsc):
    cid = lax.axis_index("c")
    sid = lax.axis_index("s")
    wid = cid * NS + sid
    ebase = wid * EW

    pltpu.sync_copy(cvec_hbm, cv_v)
    cv = cv_v[...]

    l1 = pltpu.async_copy(srci_hbm.at[pl.ds(ebase, EW)], sall, semi)
    l2 = pltpu.async_copy(dsti_hbm.at[pl.ds(ebase, EW)], dall, semg[0])

    def _z(i, _):
        szero[pl.ds(i * 16, 16)] = jnp.zeros((16,), jnp.float32)
        return 0
    lax.fori_loop(0, RPW // 16, _z, 0)
    pltpu.sync_copy(szero, sd_sh.at[pl.ds(sid * RPW, RPW)])
    pltpu.sync_copy(szero, ss_sh.at[pl.ds(sid * RPW, RPW)])
    plsc.subcore_barrier()
    l1.wait()
    l2.wait()

    def _chunk(j, _):
        cb = j * CE
        for g in range(CE // 16):
            so = pl.ds(g * 16, 16)
            bo = pl.ds(cb + g * 16, 16)
            isrc[0][so] = sall[bo]
            dsc[0][so] = dall[bo]
        d1 = pltpu.async_copy(asrc_hbm.at[isrc[0]], vas[0], semg[0])
        d2 = pltpu.async_copy(adst_hbm.at[dsc[0]], vad[0], semg[1])
        d1.wait()
        d2.wait()
        for g in range(CE // 16):
            so = pl.ds(g * 16, 16)
            v = vas[0][so] + vad[0][so]
            e = jnp.where(v >= 0.0, v, v * jnp.float32(0.2))
            vex[0][so] = jnp.exp(e - cv)
        pltpu.sync_copy(vex[0], ex_hbm.at[pl.ds(ebase + cb, CE)])
        pltpu.sync_copy(vex[0], sd_sh.at[dsc[0]], add=True)
        pltpu.sync_copy(vex[0], ss_sh.at[isrc[0]], add=True)
        return 0

    lax.fori_loop(0, NCH, _chunk, 0)
    plsc.subcore_barrier()
    sl = pl.ds(sid * RPW, RPW)
    pltpu.sync_copy(sd_sh.at[sl], sd_hbm.at[cid, sl])
    pltpu.sync_copy(ss_sh.at[sl], ss_hbm.at[cid, sl])


_stats_call = functools.partial(
    pl.kernel,
    out_type=(
        jax.ShapeDtypeStruct((E,), jnp.float32),
        jax.ShapeDtypeStruct((NC, NP), jnp.float32),
        jax.ShapeDtypeStruct((NC, NP), jnp.float32),
    ),
    mesh=_mesh,
    scratch_types=(
        pltpu.VMEM((EW,), jnp.int32),
        pltpu.VMEM((EW,), jnp.int32),
        [pltpu.VMEM((CE,), jnp.int32)] * 1,
        [pltpu.VMEM((CE,), jnp.int32)] * 1,
        [pltpu.VMEM((CE,), jnp.float32)] * 1,
        [pltpu.VMEM((CE,), jnp.float32)] * 1,
        [pltpu.VMEM((CE,), jnp.float32)] * 1,
        pltpu.VMEM((16,), jnp.float32),
        pltpu.VMEM((RPW,), jnp.float32),
        pltpu.VMEM_SHARED((NP,), jnp.float32),
        pltpu.VMEM_SHARED((NP,), jnp.float32),
        pltpu.SemaphoreType.DMA,
        [pltpu.SemaphoreType.DMA] * 2,
        pltpu.SemaphoreType.DMA,
    ),
)(_stats_body)


def _pq_body(sd_ref, ss_ref, p_ref, q_ref):
    sd = sd_ref[0] + sd_ref[1]
    ss = ss_ref[0] + ss_ref[1]
    p_ref[...] = lax.rsqrt(jnp.maximum(sd, jnp.float32(1e-30)))
    q_ref[...] = lax.rsqrt(jnp.maximum(ss, jnp.float32(1e-30)))


def _pq(sd3, ss3):
    return pl.pallas_call(
        _pq_body,
        out_shape=[
            jax.ShapeDtypeStruct((NP // 128, 128), jnp.float32),
            jax.ShapeDtypeStruct((NP // 128, 128), jnp.float32),
        ],
    )(sd3, ss3)


def _scale_body(f_ref, s_ref, o_ref):
    o_ref[...] = f_ref[...] * s_ref[...]


def _scale_rows(feat, col):
    bp = 1024
    return pl.pallas_call(
        _scale_body,
        grid=(NP // bp,),
        in_specs=[
            pl.BlockSpec((bp, 128), lambda i: (i, 0)),
            pl.BlockSpec((bp, 1), lambda i: (i, 0)),
        ],
        out_specs=pl.BlockSpec((bp, 128), lambda i: (i, 0)),
        out_shape=jax.ShapeDtypeStruct((NP, 128), jnp.float32),
    )(feat, col)


def _merge_body(pt_ref, p_ref, q_ref, h_ref, g_ref):
    h = (pt_ref[0] + pt_ref[1]) * p_ref[...]
    h_ref[...] = h
    g_ref[...] = h * q_ref[...]


def _merge(part, p_col, q_col):
    bp = 1024
    return pl.pallas_call(
        _merge_body,
        grid=(NP // bp,),
        in_specs=[
            pl.BlockSpec((NC, bp, 128), lambda i: (0, i, 0)),
            pl.BlockSpec((bp, 1), lambda i: (i, 0)),
            pl.BlockSpec((bp, 1), lambda i: (i, 0)),
        ],
        out_specs=[
            pl.BlockSpec((bp, 128), lambda i: (i, 0)),
            pl.BlockSpec((bp, 128), lambda i: (i, 0)),
        ],
        out_shape=[
            jax.ShapeDtypeStruct((NP, 128), jnp.float32),
            jax.ShapeDtypeStruct((NP, 128), jnp.float32),
        ],
    )(part, p_col, q_col)


def _prop_body(g_hbm, w_hbm, srci_hbm, dsti_hbm, out_hbm,
               sall, dall, isrc, dsc, wv, rows, zbuf, acc_sh,
               semi, semg, semw, semsc):
    cid = lax.axis_index("c")
    sid = lax.axis_index("s")
    wid = cid * NS + sid
    ebase = wid * EW

    l1 = pltpu.async_copy(srci_hbm.at[pl.ds(ebase, EW)], sall, semi)
    l2 = pltpu.async_copy(dsti_hbm.at[pl.ds(ebase, EW)], dall, semg[0])

    def _z(i, _):
        for t in range(8):
            zbuf[i, pl.ds(t * 16, 16)] = jnp.zeros((16,), jnp.float32)
        return 0
    lax.fori_loop(0, 8, _z, 0)

    def _zc(i, _):
        pltpu.sync_copy(zbuf, acc_sh.at[pl.ds(sid * RPW + i * 8, 8), :])
        return 0
    lax.fori_loop(0, RPW // 8, _zc, 0)
    plsc.subcore_barrier()
    l1.wait()
    l2.wait()

    def _issue(j, b):
        cb = j * CE
        for g in range(CE // 16):
            so = pl.ds(g * 16, 16)
            bo = pl.ds(cb + g * 16, 16)
            isrc[b][so] = sall[bo]
            dsc[b][so] = dall[bo]
        dg = pltpu.async_copy(g_hbm.at[isrc[b]], rows[b], semg[b])
        dw = pltpu.async_copy(w_hbm.at[pl.ds(ebase + cb, CE)], wv[b],
                              semw[b])
        return dg, dw

    def _finish(b, dg, dw):
        dg.wait()
        dw.wait()

        def _mul(g, _):
            w16 = wv[b][pl.ds(g * 16, 16)]
            for u in range(16):
                s = w16[u]
                for t in range(8):
                    sl = pl.ds(t * 16, 16)
                    rows[b][g * 16 + u, sl] = rows[b][g * 16 + u, sl] * s
            return 0
        lax.fori_loop(0, CE // 16, _mul, 0)
        return pltpu.async_copy(rows[b], acc_sh.at[dsc[b]], semsc,
                                add=True)

    def _pair(jj, _):
        d0 = _issue(jj * 2, 0)
        d1 = _issue(jj * 2 + 1, 1)
        s0 = _finish(0, *d0)
        s1 = _finish(1, *d1)
        s0.wait()
        s1.wait()
        return 0

    lax.fori_loop(0, NCH // 2, _pair, 0)
    dt = _issue(NCH - 1, 0)
    _finish(0, *dt).wait()

    plsc.subcore_barrier()
    sl = pl.ds(sid * RPW, RPW)
    pltpu.sync_copy(acc_sh.at[sl, :], out_hbm.at[cid, sl, :])


_prop_call = functools.partial(
    pl.kernel,
    out_type=jax.ShapeDtypeStruct((NC, NP, 128), jnp.float32),
    mesh=_mesh,
    scratch_types=(
        pltpu.VMEM((EW,), jnp.int32),
        pltpu.VMEM((EW,), jnp.int32),
        [pltpu.VMEM((CE,), jnp.int32)] * 2,
        [pltpu.VMEM((CE,), jnp.int32)] * 2,
        [pltpu.VMEM((CE,), jnp.float32)] * 2,
        [pltpu.VMEM((CE, 128), jnp.float32)] * 2,
        pltpu.VMEM((8, 128), jnp.float32),
        pltpu.VMEM_SHARED((NP, 128), jnp.float32),
        pltpu.SemaphoreType.DMA,
        [pltpu.SemaphoreType.DMA] * 2,
        [pltpu.SemaphoreType.DMA] * 2,
        pltpu.SemaphoreType.DMA,
    ),
)(_prop_body)


def _final_body(h1_ref, h2_ref, h3_ref, fd_ref, c_ref, o_ref):
    cst = c_ref[...]
    hts = []
    for k, href in enumerate((h1_ref, h2_ref, h3_ref)):
        h = href[...]
        mean = jnp.mean(h, axis=1, keepdims=True)
        var = jnp.mean(jnp.square(h - mean), axis=1, keepdims=True) \
            + jnp.float32(1e-9)
        ht = (h - mean) * cst[k:k + 1, :] * lax.rsqrt(var) \
            + cst[3 + k:4 + k, :] + cst[6 + k:7 + k, :]
        hts.append(ht)
    hop_l = cst[9:10, :]
    hop_r = cst[10:11, :]
    a_l = jnp.sum(hts[0] * hop_l, axis=1, keepdims=True)
    ls = [jnp.sum(ht * hop_r, axis=1, keepdims=True) + a_l for ht in hts]
    ls = [jnp.where(l >= 0.0, l, l * jnp.float32(0.2)) for l in ls]
    m = jnp.maximum(jnp.maximum(ls[0], ls[1]), ls[2])
    ws = [jnp.exp(l - m) for l in ls]
    tot = ws[0] + ws[1] + ws[2]
    out = fd_ref[...]
    for ht, w in zip(hts, ws):
        out = out + ht * (w / tot)
    o_ref[...] = out


def _final(h1, h2, h3, fd, consts):
    bp = 1000
    return pl.pallas_call(
        _final_body,
        grid=(N // bp,),
        in_specs=[
            pl.BlockSpec((bp, 128), lambda i: (i, 0)),
            pl.BlockSpec((bp, 128), lambda i: (i, 0)),
            pl.BlockSpec((bp, 128), lambda i: (i, 0)),
            pl.BlockSpec((bp, 128), lambda i: (i, 0)),
            pl.BlockSpec((16, 128), lambda i: (0, 0)),
        ],
        out_specs=pl.BlockSpec((bp, 128), lambda i: (i, 0)),
        out_shape=jax.ShapeDtypeStruct((N, 128), jnp.float32),
    )(h1, h2, h3, fd, consts)


def kernel(x, edge_index, W_src, W_dst, b_dst, W_attn_src, W_attn_dst,
           scale, offset, hop_attn_l, hop_attn_r, position_emb):
    srci = edge_index[0]
    dsti = edge_index[1]
    wa_pad = jnp.concatenate(
        [W_attn_src, W_attn_dst, jnp.zeros((126, 128), jnp.float32)], axis=0)
    b_row = b_dst.reshape(1, 128)

    feat_src, feat_dst, attn, cmax = _proj(x, W_src, W_dst, wa_pad, b_row)
    asrc = attn[:, 0]
    adst = attn[:, 1]
    c_off = cmax[0, 0] + cmax[0, 1]
    c_vec = jnp.full((16,), c_off, jnp.float32)

    ex, sd2, ss2 = _stats_call(asrc, adst, srci, dsti, c_vec)

    p2, q2 = _pq(sd2.reshape(NC, NP // 128, 128),
                 ss2.reshape(NC, NP // 128, 128))
    p_col = p2.reshape(NP, 1)
    q_col = q2.reshape(NP, 1)

    feat0 = jnp.pad(feat_src, ((0, NP - N), (0, 0)))
    g = _scale_rows(feat0, q_col)

    hs = []
    for _ in range(K):
        part = _prop_call(g, ex, srci, dsti)
        h, g = _merge(part, p_col, q_col)
        hs.append(h)

    consts = jnp.concatenate([
        scale[:3, 0, 0, :],
        offset[:3, 0, 0, :],
        position_emb[:, 0, :],
        hop_attn_l.reshape(1, 128),
        hop_attn_r.reshape(1, 128),
        jnp.zeros((5, 128), jnp.float32),
    ], axis=0)

    rst = _final(hs[0][:N], hs[1][:N], hs[2][:N], feat_dst, consts)
    return rst.reshape(N, 1, D)


# speedup vs baseline: 24.4507x; 1.1118x over previous
import functools

import jax
import jax.numpy as jnp
from jax import lax
from jax.experimental import pallas as pl
from jax.experimental.pallas import tpu as pltpu
from jax.experimental.pallas import tpu_sc as plsc

N = 10000
E = 320000
D = 128
K = 3
NP = 10240
NC = 2
NS = 16
NW = NC * NS
EW = E // NW
CE = 80
NCH = EW // CE
GRP = 5
NGRP = NCH // GRP
CEP = 16
NCHP = EW // CEP
NGRPP = NCHP // GRP
RPW = NP // NS

_mesh = plsc.VectorSubcoreMesh(core_axis_name="c", subcore_axis_name="s")


def _proj_body(x_ref, ws_ref, wd_ref, wa_ref, b_ref, fs_ref, fd_ref, at_ref,
               cm_ref):
    xb = x_ref[...]
    dn = (((1,), (1,)), ((), ()))
    fs_ref[...] = lax.dot_general(xb, ws_ref[...], dn,
                                  preferred_element_type=jnp.float32)
    fd_ref[...] = lax.dot_general(xb, wd_ref[...], dn,
                                  preferred_element_type=jnp.float32) + b_ref[...]
    at = lax.dot_general(xb, wa_ref[...], dn,
                         preferred_element_type=jnp.float32)
    at_ref[...] = at
    m8 = jnp.broadcast_to(jnp.max(at, axis=0, keepdims=True), (8, 128))

    @pl.when(pl.program_id(0) == 0)
    def _():
        cm_ref[...] = m8

    @pl.when(pl.program_id(0) > 0)
    def _():
        cm_ref[...] = jnp.maximum(cm_ref[...], m8)


def _proj(x, w_src, w_dst, wa_pad, b_row):
    bp = 1000
    return pl.pallas_call(
        _proj_body,
        grid=(N // bp,),
        in_specs=[
            pl.BlockSpec((bp, 128), lambda i: (i, 0)),
            pl.BlockSpec((128, 128), lambda i: (0, 0)),
            pl.BlockSpec((128, 128), lambda i: (0, 0)),
            pl.BlockSpec((128, 128), lambda i: (0, 0)),
            pl.BlockSpec((1, 128), lambda i: (0, 0)),
        ],
        out_specs=[
            pl.BlockSpec((bp, 128), lambda i: (i, 0)),
            pl.BlockSpec((bp, 128), lambda i: (i, 0)),
            pl.BlockSpec((bp, 128), lambda i: (i, 0)),
            pl.BlockSpec((8, 128), lambda i: (0, 0)),
        ],
        out_shape=[
            jax.ShapeDtypeStruct((N, 128), jnp.float32),
            jax.ShapeDtypeStruct((N, 128), jnp.float32),
            jax.ShapeDtypeStruct((N, 128), jnp.float32),
            jax.ShapeDtypeStruct((8, 128), jnp.float32),
        ],
    )(x, w_src, w_dst, wa_pad, b_row)


def _stats_body(asrc_hbm, adst_hbm, srci_hbm, dsti_hbm, cvec_hbm,
                ex_hbm, sd_hbm, ss_hbm,
                sall, dall, isrc, dsc, vas, vad, vex, cv_v, szero,
                sd_sh, ss_sh, semi, semg, semsc):
    cid = lax.axis_index("c")
    sid = lax.axis_index("s")
    wid = cid * NS + sid
    ebase = wid * EW

    pltpu.sync_copy(cvec_hbm, cv_v)
    cv = cv_v[...]

    l1 = pltpu.async_copy(srci_hbm.at[pl.ds(ebase, EW)], sall, semi)
    l2 = pltpu.async_copy(dsti_hbm.at[pl.ds(ebase, EW)], dall, semg[0])

    def _z(i, _):
        szero[pl.ds(i * 16, 16)] = jnp.zeros((16,), jnp.float32)
        return 0
    lax.fori_loop(0, RPW // 16, _z, 0)
    pltpu.sync_copy(szero, sd_sh.at[pl.ds(sid * RPW, RPW)])
    pltpu.sync_copy(szero, ss_sh.at[pl.ds(sid * RPW, RPW)])
    plsc.subcore_barrier()
    l1.wait()
    l2.wait()

    def _issue(j, b):
        cb = j * CE
        for g in range(CE // 16):
            so = pl.ds(g * 16, 16)
            bo = pl.ds(cb + g * 16, 16)
            isrc[b][so] = sall[bo]
            dsc[b][so] = dall[bo]
        d1 = pltpu.async_copy(asrc_hbm.at[isrc[b]], vas[b], semg[b])
        d2 = pltpu.async_copy(adst_hbm.at[dsc[b]], vad[b], semg[b])
        return d1, d2

    def _finish(j, b, d1, d2):
        cb = j * CE
        d1.wait()
        d2.wait()
        for g in range(CE // 16):
            so = pl.ds(g * 16, 16)
            v = vas[b][so] + vad[b][so]
            e = jnp.where(v >= 0.0, v, v * jnp.float32(0.2))
            vex[b][so] = jnp.exp(e - cv)
        pltpu.sync_copy(vex[b], ex_hbm.at[pl.ds(ebase + cb, CE)])
        pltpu.sync_copy(vex[b], sd_sh.at[dsc[b]], add=True)
        pltpu.sync_copy(vex[b], ss_sh.at[isrc[b]], add=True)

    def _pair(jj, _):
        d0 = _issue(jj * 2, 0)
        d1 = _issue(jj * 2 + 1, 1)
        _finish(jj * 2, 0, *d0)
        _finish(jj * 2 + 1, 1, *d1)
        return 0

    lax.fori_loop(0, NCH // 2, _pair, 0)
    dt = _issue(NCH - 1, 0)
    _finish(NCH - 1, 0, *dt)
    plsc.subcore_barrier()
    sl = pl.ds(sid * RPW, RPW)
    pltpu.sync_copy(sd_sh.at[sl], sd_hbm.at[cid, sl])
    pltpu.sync_copy(ss_sh.at[sl], ss_hbm.at[cid, sl])


_stats_call = functools.partial(
    pl.kernel,
    out_type=(
        jax.ShapeDtypeStruct((E,), jnp.float32),
        jax.ShapeDtypeStruct((NC, NP), jnp.float32),
        jax.ShapeDtypeStruct((NC, NP), jnp.float32),
    ),
    mesh=_mesh,
    scratch_types=(
        pltpu.VMEM((EW,), jnp.int32),
        pltpu.VMEM((EW,), jnp.int32),
        [pltpu.VMEM((CE,), jnp.int32)] * 2,
        [pltpu.VMEM((CE,), jnp.int32)] * 2,
        [pltpu.VMEM((CE,), jnp.float32)] * 2,
        [pltpu.VMEM((CE,), jnp.float32)] * 2,
        [pltpu.VMEM((CE,), jnp.float32)] * 2,
        pltpu.VMEM((16,), jnp.float32),
        pltpu.VMEM((RPW,), jnp.float32),
        pltpu.VMEM_SHARED((NP,), jnp.float32),
        pltpu.VMEM_SHARED((NP,), jnp.float32),
        pltpu.SemaphoreType.DMA,
        [pltpu.SemaphoreType.DMA] * 2,
        pltpu.SemaphoreType.DMA,
    ),
)(_stats_body)


def _pq_body(sd_ref, ss_ref, p_ref, q_ref):
    sd = sd_ref[0] + sd_ref[1]
    ss = ss_ref[0] + ss_ref[1]
    p_ref[...] = lax.rsqrt(jnp.maximum(sd, jnp.float32(1e-30)))
    q_ref[...] = lax.rsqrt(jnp.maximum(ss, jnp.float32(1e-30)))


def _pq(sd3, ss3):
    return pl.pallas_call(
        _pq_body,
        out_shape=[
            jax.ShapeDtypeStruct((NP // 128, 128), jnp.float32),
            jax.ShapeDtypeStruct((NP // 128, 128), jnp.float32),
        ],
    )(sd3, ss3)


def _scale_body(f_ref, s_ref, o_ref):
    o_ref[...] = f_ref[...] * s_ref[...]


def _scale_rows(feat, col):
    bp = 1024
    return pl.pallas_call(
        _scale_body,
        grid=(NP // bp,),
        in_specs=[
            pl.BlockSpec((bp, 128), lambda i: (i, 0)),
            pl.BlockSpec((bp, 1), lambda i: (i, 0)),
        ],
        out_specs=pl.BlockSpec((bp, 128), lambda i: (i, 0)),
        out_shape=jax.ShapeDtypeStruct((NP, 128), jnp.float32),
    )(feat, col)


def _merge_body(pt_ref, p_ref, q_ref, h_ref, g_ref):
    h = (pt_ref[0] + pt_ref[1]) * p_ref[...]
    h_ref[...] = h
    g_ref[...] = h * q_ref[...]


def _merge(part, p_col, q_col):
    bp = 1024
    return pl.pallas_call(
        _merge_body,
        grid=(NP // bp,),
        in_specs=[
            pl.BlockSpec((NC, bp, 128), lambda i: (0, i, 0)),
            pl.BlockSpec((bp, 1), lambda i: (i, 0)),
            pl.BlockSpec((bp, 1), lambda i: (i, 0)),
        ],
        out_specs=[
            pl.BlockSpec((bp, 128), lambda i: (i, 0)),
            pl.BlockSpec((bp, 128), lambda i: (i, 0)),
        ],
        out_shape=[
            jax.ShapeDtypeStruct((NP, 128), jnp.float32),
            jax.ShapeDtypeStruct((NP, 128), jnp.float32),
        ],
    )(part, p_col, q_col)


def _prop_body(g_hbm, w_hbm, srci_hbm, dsti_hbm, out_hbm,
               sall, isrc, dsc, wv, rows, zbuf, acc_sh,
               semi, semg, semw, semsc):
    cid = lax.axis_index("c")
    sid = lax.axis_index("s")
    wid = cid * NS + sid
    ebase = wid * EW

    l1 = pltpu.async_copy(srci_hbm.at[pl.ds(ebase, EW)], sall, semi)

    def _z(i, _):
        for t in range(8):
            zbuf[i, pl.ds(t * 16, 16)] = jnp.zeros((16,), jnp.float32)
        return 0
    lax.fori_loop(0, 8, _z, 0)

    def _zc(i, _):
        pltpu.sync_copy(zbuf, acc_sh.at[pl.ds(sid * RPW + i * 8, 8), :])
        return 0
    lax.fori_loop(0, RPW // 8, _zc, 0)
    plsc.subcore_barrier()
    l1.wait()

    def _issue(j, b):
        cb = j * CE
        for g in range(CE // 16):
            so = pl.ds(g * 16, 16)
            isrc[b][so] = sall[pl.ds(cb + g * 16, 16)]
        dg = pltpu.async_copy(g_hbm.at[isrc[b]], rows[b], semg[b])
        dd = pltpu.async_copy(dsti_hbm.at[pl.ds(ebase + cb, CE)], dsc[b],
                              semw[b])
        dw = pltpu.async_copy(w_hbm.at[pl.ds(ebase + cb, CE)], wv[b],
                              semw[b])
        return dg, dd, dw

    def _finish(b, dg, dd, dw):
        dg.wait()
        dd.wait()
        dw.wait()

        def _mul(g, _):
            w16 = wv[b][pl.ds(g * 16, 16)]
            for u in range(16):
                s = w16[u]
                for t in range(8):
                    sl = pl.ds(t * 16, 16)
                    rows[b][g * 16 + u, sl] = rows[b][g * 16 + u, sl] * s
            return 0
        lax.fori_loop(0, CE // 16, _mul, 0)
        return pltpu.async_copy(rows[b], acc_sh.at[dsc[b]], semsc,
                                add=True)

    def _trip(jj, _):
        ds_ = [_issue(jj * 3 + b, b) for b in range(3)]
        ss = [_finish(b, *ds_[b]) for b in range(3)]
        for s in ss:
            s.wait()
        return 0

    lax.fori_loop(0, NCH // 3, _trip, 0)
    dt = [_issue(NCH - 2 + b, b) for b in range(2)]
    st = [_finish(b, *dt[b]) for b in range(2)]
    for s in st:
        s.wait()

    plsc.subcore_barrier()
    sl = pl.ds(sid * RPW, RPW)
    pltpu.sync_copy(acc_sh.at[sl, :], out_hbm.at[cid, sl, :])


_prop_call = functools.partial(
    pl.kernel,
    out_type=jax.ShapeDtypeStruct((NC, NP, 128), jnp.float32),
    mesh=_mesh,
    scratch_types=(
        pltpu.VMEM((EW,), jnp.int32),
        [pltpu.VMEM((CE,), jnp.int32)] * 3,
        [pltpu.VMEM((CE,), jnp.int32)] * 3,
        [pltpu.VMEM((CE,), jnp.float32)] * 3,
        [pltpu.VMEM((CE, 128), jnp.float32)] * 3,
        pltpu.VMEM((8, 128), jnp.float32),
        pltpu.VMEM_SHARED((NP, 128), jnp.float32),
        pltpu.SemaphoreType.DMA,
        [pltpu.SemaphoreType.DMA] * 3,
        [pltpu.SemaphoreType.DMA] * 3,
        pltpu.SemaphoreType.DMA,
    ),
)(_prop_body)


def _final_body(h1_ref, h2_ref, h3_ref, fd_ref, c_ref, o_ref):
    cst = c_ref[...]
    hts = []
    for k, href in enumerate((h1_ref, h2_ref, h3_ref)):
        h = href[...]
        mean = jnp.mean(h, axis=1, keepdims=True)
        var = jnp.mean(jnp.square(h - mean), axis=1, keepdims=True) \
            + jnp.float32(1e-9)
        ht = (h - mean) * cst[k:k + 1, :] * lax.rsqrt(var) \
            + cst[3 + k:4 + k, :] + cst[6 + k:7 + k, :]
        hts.append(ht)
    hop_l = cst[9:10, :]
    hop_r = cst[10:11, :]
    a_l = jnp.sum(hts[0] * hop_l, axis=1, keepdims=True)
    ls = [jnp.sum(ht * hop_r, axis=1, keepdims=True) + a_l for ht in hts]
    ls = [jnp.where(l >= 0.0, l, l * jnp.float32(0.2)) for l in ls]
    m = jnp.maximum(jnp.maximum(ls[0], ls[1]), ls[2])
    ws = [jnp.exp(l - m) for l in ls]
    tot = ws[0] + ws[1] + ws[2]
    out = fd_ref[...]
    for ht, w in zip(hts, ws):
        out = out + ht * (w / tot)
    o_ref[...] = out


def _final(h1, h2, h3, fd, consts):
    bp = 1000
    return pl.pallas_call(
        _final_body,
        grid=(N // bp,),
        in_specs=[
            pl.BlockSpec((bp, 128), lambda i: (i, 0)),
            pl.BlockSpec((bp, 128), lambda i: (i, 0)),
            pl.BlockSpec((bp, 128), lambda i: (i, 0)),
            pl.BlockSpec((bp, 128), lambda i: (i, 0)),
            pl.BlockSpec((16, 128), lambda i: (0, 0)),
        ],
        out_specs=pl.BlockSpec((bp, 128), lambda i: (i, 0)),
        out_shape=jax.ShapeDtypeStruct((N, 128), jnp.float32),
    )(h1, h2, h3, fd, consts)


def kernel(x, edge_index, W_src, W_dst, b_dst, W_attn_src, W_attn_dst,
           scale, offset, hop_attn_l, hop_attn_r, position_emb):
    srci = edge_index[0]
    dsti = edge_index[1]
    wa_pad = jnp.concatenate(
        [W_attn_src, W_attn_dst, jnp.zeros((126, 128), jnp.float32)], axis=0)
    b_row = b_dst.reshape(1, 128)

    feat_src, feat_dst, attn, cmax = _proj(x, W_src, W_dst, wa_pad, b_row)
    asrc = attn[:, 0]
    adst = attn[:, 1]
    c_off = cmax[0, 0] + cmax[0, 1]
    c_vec = jnp.full((16,), c_off, jnp.float32)

    ex, sd2, ss2 = _stats_call(asrc, adst, srci, dsti, c_vec)

    p2, q2 = _pq(sd2.reshape(NC, NP // 128, 128),
                 ss2.reshape(NC, NP // 128, 128))
    p_col = p2.reshape(NP, 1)
    q_col = q2.reshape(NP, 1)

    feat0 = jnp.pad(feat_src, ((0, NP - N), (0, 0)))
    g = _scale_rows(feat0, q_col)

    hs = []
    for _ in range(K):
        part = _prop_call(g, ex, srci, dsti)
        h, g = _merge(part, p_col, q_col)
        hs.append(h)

    consts = jnp.concatenate([
        scale[:3, 0, 0, :],
        offset[:3, 0, 0, :],
        position_emb[:, 0, :],
        hop_attn_l.reshape(1, 128),
        hop_attn_r.reshape(1, 128),
        jnp.zeros((5, 128), jnp.float32),
    ], axis=0)

    rst = _final(hs[0][:N], hs[1][:N], hs[2][:N], feat_dst, consts)
    return rst.reshape(N, 1, D)


# speedup vs baseline: 24.9442x; 1.0202x over previous
import functools

import jax
import jax.numpy as jnp
from jax import lax
from jax.experimental import pallas as pl
from jax.experimental.pallas import tpu as pltpu
from jax.experimental.pallas import tpu_sc as plsc

N = 10000
E = 320000
D = 128
K = 3
NP = 10240
NC = 2
NS = 16
NW = NC * NS
EW = E // NW
CE = 80
NCH = EW // CE
GRP = 5
NGRP = NCH // GRP
CEP = 16
NCHP = EW // CEP
NGRPP = NCHP // GRP
RPW = NP // NS

_mesh = plsc.VectorSubcoreMesh(core_axis_name="c", subcore_axis_name="s")


def _proj_body(x_ref, ws_ref, wd_ref, wa_ref, b_ref, fs_ref, fd_ref, at_ref,
               cm_ref):
    xb = x_ref[...]
    dn = (((1,), (1,)), ((), ()))
    fs_ref[...] = lax.dot_general(xb, ws_ref[...], dn,
                                  preferred_element_type=jnp.float32)
    fd_ref[...] = lax.dot_general(xb, wd_ref[...], dn,
                                  preferred_element_type=jnp.float32) + b_ref[...]
    at = lax.dot_general(xb, wa_ref[...], dn,
                         preferred_element_type=jnp.float32)
    at_ref[...] = at
    m8 = jnp.broadcast_to(jnp.max(at, axis=0, keepdims=True), (8, 128))

    @pl.when(pl.program_id(0) == 0)
    def _():
        cm_ref[...] = m8

    @pl.when(pl.program_id(0) > 0)
    def _():
        cm_ref[...] = jnp.maximum(cm_ref[...], m8)


def _proj(x, w_src, w_dst, wa_pad, b_row):
    bp = 1000
    return pl.pallas_call(
        _proj_body,
        grid=(N // bp,),
        in_specs=[
            pl.BlockSpec((bp, 128), lambda i: (i, 0)),
            pl.BlockSpec((128, 128), lambda i: (0, 0)),
            pl.BlockSpec((128, 128), lambda i: (0, 0)),
            pl.BlockSpec((128, 128), lambda i: (0, 0)),
            pl.BlockSpec((1, 128), lambda i: (0, 0)),
        ],
        out_specs=[
            pl.BlockSpec((bp, 128), lambda i: (i, 0)),
            pl.BlockSpec((bp, 128), lambda i: (i, 0)),
            pl.BlockSpec((bp, 128), lambda i: (i, 0)),
            pl.BlockSpec((8, 128), lambda i: (0, 0)),
        ],
        out_shape=[
            jax.ShapeDtypeStruct((N, 128), jnp.float32),
            jax.ShapeDtypeStruct((N, 128), jnp.float32),
            jax.ShapeDtypeStruct((N, 128), jnp.float32),
            jax.ShapeDtypeStruct((8, 128), jnp.float32),
        ],
    )(x, w_src, w_dst, wa_pad, b_row)


def _stats_body(asrc_hbm, adst_hbm, srci_hbm, dsti_hbm, cvec_hbm,
                ex_hbm, sd_hbm, ss_hbm,
                sall, dall, isrc, dsc, vas, vad, vex, cv_v, szero,
                sd_sh, ss_sh, semi, semg, semsc):
    cid = lax.axis_index("c")
    sid = lax.axis_index("s")
    wid = cid * NS + sid
    ebase = wid * EW

    pltpu.sync_copy(cvec_hbm, cv_v)
    cv = cv_v[...]

    l1 = pltpu.async_copy(srci_hbm.at[pl.ds(ebase, EW)], sall, semi)
    l2 = pltpu.async_copy(dsti_hbm.at[pl.ds(ebase, EW)], dall, semg[0])

    def _z(i, _):
        szero[pl.ds(i * 16, 16)] = jnp.zeros((16,), jnp.float32)
        return 0
    lax.fori_loop(0, RPW // 16, _z, 0)
    pltpu.sync_copy(szero, sd_sh.at[pl.ds(sid * RPW, RPW)])
    pltpu.sync_copy(szero, ss_sh.at[pl.ds(sid * RPW, RPW)])
    plsc.subcore_barrier()
    l1.wait()
    l2.wait()

    def _issue(j, b):
        cb = j * CE
        for g in range(CE // 16):
            so = pl.ds(g * 16, 16)
            bo = pl.ds(cb + g * 16, 16)
            isrc[b][so] = sall[bo]
            dsc[b][so] = dall[bo]
        d1 = pltpu.async_copy(asrc_hbm.at[isrc[b]], vas[b], semg[b])
        d2 = pltpu.async_copy(adst_hbm.at[dsc[b]], vad[b], semg[b])
        return d1, d2

    def _finish(j, b, d1, d2):
        cb = j * CE
        d1.wait()
        d2.wait()
        for g in range(CE // 16):
            so = pl.ds(g * 16, 16)
            v = vas[b][so] + vad[b][so]
            e = jnp.where(v >= 0.0, v, v * jnp.float32(0.2))
            vex[b][so] = jnp.exp(e - cv)
        pltpu.sync_copy(vex[b], ex_hbm.at[pl.ds(ebase + cb, CE)])
        s1 = pltpu.async_copy(vex[b], sd_sh.at[dsc[b]], semsc[b],
                              add=True)
        s2 = pltpu.async_copy(vex[b], ss_sh.at[isrc[b]], semsc[b],
                              add=True)
        return s1, s2

    def _pair(jj, _):
        d0 = _issue(jj * 2, 0)
        d1 = _issue(jj * 2 + 1, 1)
        s0 = _finish(jj * 2, 0, *d0)
        s1 = _finish(jj * 2 + 1, 1, *d1)
        for s in s0 + s1:
            s.wait()
        return 0

    lax.fori_loop(0, NCH // 2, _pair, 0)
    dt = _issue(NCH - 1, 0)
    for s in _finish(NCH - 1, 0, *dt):
        s.wait()
    plsc.subcore_barrier()
    sl = pl.ds(sid * RPW, RPW)
    pltpu.sync_copy(sd_sh.at[sl], sd_hbm.at[cid, sl])
    pltpu.sync_copy(ss_sh.at[sl], ss_hbm.at[cid, sl])


_stats_call = functools.partial(
    pl.kernel,
    out_type=(
        jax.ShapeDtypeStruct((E,), jnp.float32),
        jax.ShapeDtypeStruct((NC, NP), jnp.float32),
        jax.ShapeDtypeStruct((NC, NP), jnp.float32),
    ),
    mesh=_mesh,
    scratch_types=(
        pltpu.VMEM((EW,), jnp.int32),
        pltpu.VMEM((EW,), jnp.int32),
        [pltpu.VMEM((CE,), jnp.int32)] * 2,
        [pltpu.VMEM((CE,), jnp.int32)] * 2,
        [pltpu.VMEM((CE,), jnp.float32)] * 2,
        [pltpu.VMEM((CE,), jnp.float32)] * 2,
        [pltpu.VMEM((CE,), jnp.float32)] * 2,
        pltpu.VMEM((16,), jnp.float32),
        pltpu.VMEM((RPW,), jnp.float32),
        pltpu.VMEM_SHARED((NP,), jnp.float32),
        pltpu.VMEM_SHARED((NP,), jnp.float32),
        pltpu.SemaphoreType.DMA,
        [pltpu.SemaphoreType.DMA] * 2,
        [pltpu.SemaphoreType.DMA] * 2,
    ),
)(_stats_body)


def _pq_body(sd_ref, ss_ref, p_ref, q_ref):
    sd = sd_ref[0] + sd_ref[1]
    ss = ss_ref[0] + ss_ref[1]
    p_ref[...] = lax.rsqrt(jnp.maximum(sd, jnp.float32(1e-30)))
    q_ref[...] = lax.rsqrt(jnp.maximum(ss, jnp.float32(1e-30)))


def _pq(sd3, ss3):
    return pl.pallas_call(
        _pq_body,
        out_shape=[
            jax.ShapeDtypeStruct((NP // 128, 128), jnp.float32),
            jax.ShapeDtypeStruct((NP // 128, 128), jnp.float32),
        ],
    )(sd3, ss3)


def _scale_body(f_ref, s_ref, o_ref):
    o_ref[...] = f_ref[...] * s_ref[...]


def _scale_rows(feat, col):
    bp = 1024
    return pl.pallas_call(
        _scale_body,
        grid=(NP // bp,),
        in_specs=[
            pl.BlockSpec((bp, 128), lambda i: (i, 0)),
            pl.BlockSpec((bp, 1), lambda i: (i, 0)),
        ],
        out_specs=pl.BlockSpec((bp, 128), lambda i: (i, 0)),
        out_shape=jax.ShapeDtypeStruct((NP, 128), jnp.float32),
    )(feat, col)


def _merge_body(pt_ref, p_ref, q_ref, h_ref, g_ref):
    h = (pt_ref[0] + pt_ref[1]) * p_ref[...]
    h_ref[...] = h
    g_ref[...] = h * q_ref[...]


def _merge(part, p_col, q_col):
    bp = 1024
    return pl.pallas_call(
        _merge_body,
        grid=(NP // bp,),
        in_specs=[
            pl.BlockSpec((NC, bp, 128), lambda i: (0, i, 0)),
            pl.BlockSpec((bp, 1), lambda i: (i, 0)),
            pl.BlockSpec((bp, 1), lambda i: (i, 0)),
        ],
        out_specs=[
            pl.BlockSpec((bp, 128), lambda i: (i, 0)),
            pl.BlockSpec((bp, 128), lambda i: (i, 0)),
        ],
        out_shape=[
            jax.ShapeDtypeStruct((NP, 128), jnp.float32),
            jax.ShapeDtypeStruct((NP, 128), jnp.float32),
        ],
    )(part, p_col, q_col)


def _prop_body(g_hbm, w_hbm, srci_hbm, dsti_hbm, out_hbm,
               sall, isrc, dsc, wv, rows, zbuf, acc_sh,
               semi, semg, semw, semsc):
    cid = lax.axis_index("c")
    sid = lax.axis_index("s")
    wid = cid * NS + sid
    ebase = wid * EW

    l1 = pltpu.async_copy(srci_hbm.at[pl.ds(ebase, EW)], sall, semi)

    def _z(i, _):
        for t in range(8):
            zbuf[i, pl.ds(t * 16, 16)] = jnp.zeros((16,), jnp.float32)
        return 0
    lax.fori_loop(0, 8, _z, 0)

    def _zc(i, _):
        pltpu.sync_copy(zbuf, acc_sh.at[pl.ds(sid * RPW + i * 8, 8), :])
        return 0
    lax.fori_loop(0, RPW // 8, _zc, 0)
    plsc.subcore_barrier()
    l1.wait()

    def _issue(j, b):
        cb = j * CE
        for g in range(CE // 16):
            so = pl.ds(g * 16, 16)
            isrc[b][so] = sall[pl.ds(cb + g * 16, 16)]
        dg = pltpu.async_copy(g_hbm.at[isrc[b]], rows[b], semg[b])
        dd = pltpu.async_copy(dsti_hbm.at[pl.ds(ebase + cb, CE)], dsc[b],
                              semw[b])
        dw = pltpu.async_copy(w_hbm.at[pl.ds(ebase + cb, CE)], wv[b],
                              semw[b])
        return dg, dd, dw

    def _finish(b, dg, dd, dw):
        dg.wait()
        dd.wait()
        dw.wait()

        def _mul(g, _):
            w16 = wv[b][pl.ds(g * 16, 16)]
            for u in range(16):
                s = w16[u]
                for t in range(8):
                    sl = pl.ds(t * 16, 16)
                    rows[b][g * 16 + u, sl] = rows[b][g * 16 + u, sl] * s
            return 0
        lax.fori_loop(0, CE // 16, _mul, 0)
        return pltpu.async_copy(rows[b], acc_sh.at[dsc[b]], semsc,
                                add=True)

    def _trip(jj, _):
        ds_ = [_issue(jj * 3 + b, b) for b in range(3)]
        ss = [_finish(b, *ds_[b]) for b in range(3)]
        for s in ss:
            s.wait()
        return 0

    lax.fori_loop(0, NCH // 3, _trip, 0)
    dt = [_issue(NCH - 2 + b, b) for b in range(2)]
    st = [_finish(b, *dt[b]) for b in range(2)]
    for s in st:
        s.wait()

    plsc.subcore_barrier()
    sl = pl.ds(sid * RPW, RPW)
    pltpu.sync_copy(acc_sh.at[sl, :], out_hbm.at[cid, sl, :])


_prop_call = functools.partial(
    pl.kernel,
    out_type=jax.ShapeDtypeStruct((NC, NP, 128), jnp.float32),
    mesh=_mesh,
    scratch_types=(
        pltpu.VMEM((EW,), jnp.int32),
        [pltpu.VMEM((CE,), jnp.int32)] * 3,
        [pltpu.VMEM((CE,), jnp.int32)] * 3,
        [pltpu.VMEM((CE,), jnp.float32)] * 3,
        [pltpu.VMEM((CE, 128), jnp.float32)] * 3,
        pltpu.VMEM((8, 128), jnp.float32),
        pltpu.VMEM_SHARED((NP, 128), jnp.float32),
        pltpu.SemaphoreType.DMA,
        [pltpu.SemaphoreType.DMA] * 3,
        [pltpu.SemaphoreType.DMA] * 3,
        pltpu.SemaphoreType.DMA,
    ),
)(_prop_body)


def _final_body(h1_ref, h2_ref, h3_ref, fd_ref, c_ref, o_ref):
    cst = c_ref[...]
    hts = []
    for k, href in enumerate((h1_ref, h2_ref, h3_ref)):
        h = href[...]
        mean = jnp.mean(h, axis=1, keepdims=True)
        var = jnp.mean(jnp.square(h - mean), axis=1, keepdims=True) \
            + jnp.float32(1e-9)
        ht = (h - mean) * cst[k:k + 1, :] * lax.rsqrt(var) \
            + cst[3 + k:4 + k, :] + cst[6 + k:7 + k, :]
        hts.append(ht)
    hop_l = cst[9:10, :]
    hop_r = cst[10:11, :]
    a_l = jnp.sum(hts[0] * hop_l, axis=1, keepdims=True)
    ls = [jnp.sum(ht * hop_r, axis=1, keepdims=True) + a_l for ht in hts]
    ls = [jnp.where(l >= 0.0, l, l * jnp.float32(0.2)) for l in ls]
    m = jnp.maximum(jnp.maximum(ls[0], ls[1]), ls[2])
    ws = [jnp.exp(l - m) for l in ls]
    tot = ws[0] + ws[1] + ws[2]
    out = fd_ref[...]
    for ht, w in zip(hts, ws):
        out = out + ht * (w / tot)
    o_ref[...] = out


def _final(h1, h2, h3, fd, consts):
    bp = 1000
    return pl.pallas_call(
        _final_body,
        grid=(N // bp,),
        in_specs=[
            pl.BlockSpec((bp, 128), lambda i: (i, 0)),
            pl.BlockSpec((bp, 128), lambda i: (i, 0)),
            pl.BlockSpec((bp, 128), lambda i: (i, 0)),
            pl.BlockSpec((bp, 128), lambda i: (i, 0)),
            pl.BlockSpec((16, 128), lambda i: (0, 0)),
        ],
        out_specs=pl.BlockSpec((bp, 128), lambda i: (i, 0)),
        out_shape=jax.ShapeDtypeStruct((N, 128), jnp.float32),
    )(h1, h2, h3, fd, consts)


def kernel(x, edge_index, W_src, W_dst, b_dst, W_attn_src, W_attn_dst,
           scale, offset, hop_attn_l, hop_attn_r, position_emb):
    srci = edge_index[0]
    dsti = edge_index[1]
    wa_pad = jnp.concatenate(
        [W_attn_src, W_attn_dst, jnp.zeros((126, 128), jnp.float32)], axis=0)
    b_row = b_dst.reshape(1, 128)

    feat_src, feat_dst, attn, cmax = _proj(x, W_src, W_dst, wa_pad, b_row)
    asrc = attn[:, 0]
    adst = attn[:, 1]
    c_off = cmax[0, 0] + cmax[0, 1]
    c_vec = jnp.full((16,), c_off, jnp.float32)

    ex, sd2, ss2 = _stats_call(asrc, adst, srci, dsti, c_vec)

    p2, q2 = _pq(sd2.reshape(NC, NP // 128, 128),
                 ss2.reshape(NC, NP // 128, 128))
    p_col = p2.reshape(NP, 1)
    q_col = q2.reshape(NP, 1)

    feat0 = jnp.pad(feat_src, ((0, NP - N), (0, 0)))
    g = _scale_rows(feat0, q_col)

    hs = []
    for _ in range(K):
        part = _prop_call(g, ex, srci, dsti)
        h, g = _merge(part, p_col, q_col)
        hs.append(h)

    consts = jnp.concatenate([
        scale[:3, 0, 0, :],
        offset[:3, 0, 0, :],
        position_emb[:, 0, :],
        hop_attn_l.reshape(1, 128),
        hop_attn_r.reshape(1, 128),
        jnp.zeros((5, 128), jnp.float32),
    ], axis=0)

    rst = _final(hs[0][:N], hs[1][:N], hs[2][:N], feat_dst, consts)
    return rst.reshape(N, 1, D)


# speedup vs baseline: 30.3154x; 1.2153x over previous
import functools

import jax
import jax.numpy as jnp
from jax import lax
from jax.experimental import pallas as pl
from jax.experimental.pallas import tpu as pltpu
from jax.experimental.pallas import tpu_sc as plsc

N = 10000
E = 320000
D = 128
K = 3
NP = 10240
NC = 2
NS = 16
NW = NC * NS
EW = E // NW
CE = 80
NCH = EW // CE
GRP = 5
NGRP = NCH // GRP
CEP = 16
NCHP = EW // CEP
NGRPP = NCHP // GRP
RPW = NP // NS

_mesh = plsc.VectorSubcoreMesh(core_axis_name="c", subcore_axis_name="s")


def _proj_body(x_ref, ws_ref, wd_ref, wa_ref, b_ref, fs_ref, fd_ref, at_ref,
               cm_ref):
    xb = x_ref[...]
    dn = (((1,), (1,)), ((), ()))
    fs_ref[...] = lax.dot_general(xb, ws_ref[...], dn,
                                  preferred_element_type=jnp.float32)
    fd_ref[...] = lax.dot_general(xb, wd_ref[...], dn,
                                  preferred_element_type=jnp.float32) + b_ref[...]
    at = lax.dot_general(xb, wa_ref[...], dn,
                         preferred_element_type=jnp.float32)
    at_ref[...] = at
    m8 = jnp.broadcast_to(jnp.max(at, axis=0, keepdims=True), (8, 128))

    @pl.when(pl.program_id(0) == 0)
    def _():
        cm_ref[...] = m8

    @pl.when(pl.program_id(0) > 0)
    def _():
        cm_ref[...] = jnp.maximum(cm_ref[...], m8)


def _proj(x, w_src, w_dst, wa_pad, b_row):
    bp = 1000
    return pl.pallas_call(
        _proj_body,
        grid=(N // bp,),
        in_specs=[
            pl.BlockSpec((bp, 128), lambda i: (i, 0)),
            pl.BlockSpec((128, 128), lambda i: (0, 0)),
            pl.BlockSpec((128, 128), lambda i: (0, 0)),
            pl.BlockSpec((128, 128), lambda i: (0, 0)),
            pl.BlockSpec((1, 128), lambda i: (0, 0)),
        ],
        out_specs=[
            pl.BlockSpec((bp, 128), lambda i: (i, 0)),
            pl.BlockSpec((bp, 128), lambda i: (i, 0)),
            pl.BlockSpec((bp, 128), lambda i: (i, 0)),
            pl.BlockSpec((8, 128), lambda i: (0, 0)),
        ],
        out_shape=[
            jax.ShapeDtypeStruct((N, 128), jnp.float32),
            jax.ShapeDtypeStruct((N, 128), jnp.float32),
            jax.ShapeDtypeStruct((N, 128), jnp.float32),
            jax.ShapeDtypeStruct((8, 128), jnp.float32),
        ],
    )(x, w_src, w_dst, wa_pad, b_row)


def _stats_body(asrc_hbm, adst_hbm, srci_hbm, dsti_hbm, cvec_hbm,
                ex_hbm, sd_hbm, ss_hbm,
                sall, dall, isrc, dsc, vas, vad, vex, cv_v, szero,
                sd_sh, ss_sh, semi, semg, semsc):
    cid = lax.axis_index("c")
    sid = lax.axis_index("s")
    wid = cid * NS + sid
    ebase = wid * EW

    pltpu.sync_copy(cvec_hbm, cv_v)
    cv = cv_v[...]

    l1 = pltpu.async_copy(srci_hbm.at[pl.ds(ebase, EW)], sall, semi)
    l2 = pltpu.async_copy(dsti_hbm.at[pl.ds(ebase, EW)], dall, semg[0])

    def _z(i, _):
        szero[pl.ds(i * 16, 16)] = jnp.zeros((16,), jnp.float32)
        return 0
    lax.fori_loop(0, RPW // 16, _z, 0)
    pltpu.sync_copy(szero, sd_sh.at[pl.ds(sid * RPW, RPW)])
    pltpu.sync_copy(szero, ss_sh.at[pl.ds(sid * RPW, RPW)])
    plsc.subcore_barrier()
    l1.wait()
    l2.wait()

    def _issue(j, b):
        cb = j * CE
        for g in range(CE // 16):
            so = pl.ds(g * 16, 16)
            bo = pl.ds(cb + g * 16, 16)
            isrc[b][so] = sall[bo]
            dsc[b][so] = dall[bo]
        d1 = pltpu.async_copy(asrc_hbm.at[isrc[b]], vas[b], semg[b])
        d2 = pltpu.async_copy(adst_hbm.at[dsc[b]], vad[b], semg[b])
        return d1, d2

    def _finish(j, b, d1, d2):
        cb = j * CE
        d1.wait()
        d2.wait()
        for g in range(CE // 16):
            so = pl.ds(g * 16, 16)
            v = vas[b][so] + vad[b][so]
            e = jnp.where(v >= 0.0, v, v * jnp.float32(0.2))
            vex[b][so] = jnp.exp(e - cv)
        pltpu.sync_copy(vex[b], ex_hbm.at[pl.ds(ebase + cb, CE)])
        s1 = pltpu.async_copy(vex[b], sd_sh.at[dsc[b]], semsc[b],
                              add=True)
        s2 = pltpu.async_copy(vex[b], ss_sh.at[isrc[b]], semsc[b],
                              add=True)
        return s1, s2

    def _pair(jj, _):
        d0 = _issue(jj * 2, 0)
        d1 = _issue(jj * 2 + 1, 1)
        s0 = _finish(jj * 2, 0, *d0)
        s1 = _finish(jj * 2 + 1, 1, *d1)
        for s in s0 + s1:
            s.wait()
        return 0

    lax.fori_loop(0, NCH // 2, _pair, 0)
    dt = _issue(NCH - 1, 0)
    for s in _finish(NCH - 1, 0, *dt):
        s.wait()
    plsc.subcore_barrier()
    sl = pl.ds(sid * RPW, RPW)
    pltpu.sync_copy(sd_sh.at[sl], sd_hbm.at[cid, sl])
    pltpu.sync_copy(ss_sh.at[sl], ss_hbm.at[cid, sl])


_stats_call = functools.partial(
    pl.kernel,
    out_type=(
        jax.ShapeDtypeStruct((E,), jnp.float32),
        jax.ShapeDtypeStruct((NC, NP), jnp.float32),
        jax.ShapeDtypeStruct((NC, NP), jnp.float32),
    ),
    mesh=_mesh,
    scratch_types=(
        pltpu.VMEM((EW,), jnp.int32),
        pltpu.VMEM((EW,), jnp.int32),
        [pltpu.VMEM((CE,), jnp.int32)] * 2,
        [pltpu.VMEM((CE,), jnp.int32)] * 2,
        [pltpu.VMEM((CE,), jnp.float32)] * 2,
        [pltpu.VMEM((CE,), jnp.float32)] * 2,
        [pltpu.VMEM((CE,), jnp.float32)] * 2,
        pltpu.VMEM((16,), jnp.float32),
        pltpu.VMEM((RPW,), jnp.float32),
        pltpu.VMEM_SHARED((NP,), jnp.float32),
        pltpu.VMEM_SHARED((NP,), jnp.float32),
        pltpu.SemaphoreType.DMA,
        [pltpu.SemaphoreType.DMA] * 2,
        [pltpu.SemaphoreType.DMA] * 2,
    ),
)(_stats_body)


def _pq_body(sd_ref, ss_ref, p_ref, q_ref):
    sd = sd_ref[0] + sd_ref[1]
    ss = ss_ref[0] + ss_ref[1]
    p_ref[...] = lax.rsqrt(jnp.maximum(sd, jnp.float32(1e-30)))
    q_ref[...] = lax.rsqrt(jnp.maximum(ss, jnp.float32(1e-30)))


def _pq(sd3, ss3):
    return pl.pallas_call(
        _pq_body,
        out_shape=[
            jax.ShapeDtypeStruct((NP // 128, 128), jnp.float32),
            jax.ShapeDtypeStruct((NP // 128, 128), jnp.float32),
        ],
    )(sd3, ss3)


def _scale_body(f_ref, s_ref, o_ref):
    o_ref[...] = f_ref[...] * s_ref[...]


def _scale_rows(feat, col):
    bp = 1024
    return pl.pallas_call(
        _scale_body,
        grid=(NP // bp,),
        in_specs=[
            pl.BlockSpec((bp, 128), lambda i: (i, 0)),
            pl.BlockSpec((bp, 1), lambda i: (i, 0)),
        ],
        out_specs=pl.BlockSpec((bp, 128), lambda i: (i, 0)),
        out_shape=jax.ShapeDtypeStruct((NP, 128), jnp.float32),
    )(feat, col)


def _merge_body(pt_ref, p_ref, q_ref, h_ref, g_ref):
    h = (pt_ref[0] + pt_ref[1]) * p_ref[...]
    h_ref[...] = h
    g_ref[...] = h * q_ref[...]


def _merge(part, p_col, q_col):
    bp = 1024
    return pl.pallas_call(
        _merge_body,
        grid=(NP // bp,),
        in_specs=[
            pl.BlockSpec((NC, bp, 128), lambda i: (0, i, 0)),
            pl.BlockSpec((bp, 1), lambda i: (i, 0)),
            pl.BlockSpec((bp, 1), lambda i: (i, 0)),
        ],
        out_specs=[
            pl.BlockSpec((bp, 128), lambda i: (i, 0)),
            pl.BlockSpec((bp, 128), lambda i: (i, 0)),
        ],
        out_shape=[
            jax.ShapeDtypeStruct((NP, 128), jnp.float32),
            jax.ShapeDtypeStruct((NP, 128), jnp.float32),
        ],
    )(part, p_col, q_col)


def _prop_body(g_hbm, w_hbm, srci_hbm, dsti_hbm, out_hbm,
               sall, isrc, dsc, wv, rows, zbuf, acc_sh,
               semi, semg, semw, semsc):
    cid = lax.axis_index("c")
    sid = lax.axis_index("s")
    wid = cid * NS + sid
    ebase = wid * EW

    l1 = pltpu.async_copy(srci_hbm.at[pl.ds(ebase, EW)], sall, semi)

    def _z(i, _):
        for t in range(8):
            zbuf[i, pl.ds(t * 16, 16)] = jnp.zeros((16,), jnp.float32)
        return 0
    lax.fori_loop(0, 8, _z, 0)

    def _zc(i, _):
        pltpu.sync_copy(zbuf, acc_sh.at[pl.ds(sid * RPW + i * 8, 8), :])
        return 0
    lax.fori_loop(0, RPW // 8, _zc, 0)
    plsc.subcore_barrier()
    l1.wait()

    def _issue(j, b):
        cb = j * CE
        for g in range(CE // 16):
            so = pl.ds(g * 16, 16)
            isrc[b][so] = sall[pl.ds(cb + g * 16, 16)]
        pltpu.async_copy(g_hbm.at[isrc[b]], rows[b], semg[b])
        pltpu.async_copy(dsti_hbm.at[pl.ds(ebase + cb, CE)], dsc[b],
                         semw[b])
        pltpu.async_copy(w_hbm.at[pl.ds(ebase + cb, CE)], wv[b],
                         semw[b])

    def _wait_in(b):
        pltpu.make_async_copy(g_hbm.at[pl.ds(0, CE), :], rows[b],
                              semg[b]).wait()
        pltpu.make_async_copy(dsti_hbm.at[pl.ds(0, CE)], dsc[b],
                              semw[b]).wait()
        pltpu.make_async_copy(w_hbm.at[pl.ds(0, CE)], wv[b],
                              semw[b]).wait()

    def _wait_sc(b):
        pltpu.make_async_copy(g_hbm.at[pl.ds(0, CE), :], rows[b],
                              semsc[b]).wait()

    def _finish(b):
        _wait_in(b)

        def _mul(g, _):
            w16 = wv[b][pl.ds(g * 16, 16)]
            for u in range(16):
                s = w16[u]
                for t in range(8):
                    sl = pl.ds(t * 16, 16)
                    rows[b][g * 16 + u, sl] = rows[b][g * 16 + u, sl] * s
            return 0
        lax.fori_loop(0, CE // 16, _mul, 0)
        pltpu.async_copy(rows[b], acc_sh.at[dsc[b]], semsc[b], add=True)

    NB = NCH // 3
    for b in range(3):
        _issue(b, b)

    def _body(i, _):
        _finish(0)
        _finish(1)
        _wait_sc(0)
        _issue(i * 3 + 3, 0)
        _finish(2)
        _wait_sc(1)
        _issue(i * 3 + 4, 1)
        _wait_sc(2)
        _issue(i * 3 + 5, 2)
        return 0

    lax.fori_loop(0, NB - 1, _body, 0)
    _finish(0)
    _finish(1)
    _wait_sc(0)
    _issue(NCH - 2, 0)
    _finish(2)
    _wait_sc(1)
    _issue(NCH - 1, 1)
    _finish(0)
    _finish(1)
    _wait_sc(2)
    _wait_sc(0)
    _wait_sc(1)

    plsc.subcore_barrier()
    sl = pl.ds(sid * RPW, RPW)
    pltpu.sync_copy(acc_sh.at[sl, :], out_hbm.at[cid, sl, :])


_prop_call = functools.partial(
    pl.kernel,
    out_type=jax.ShapeDtypeStruct((NC, NP, 128), jnp.float32),
    mesh=_mesh,
    scratch_types=(
        pltpu.VMEM((EW,), jnp.int32),
        [pltpu.VMEM((CE,), jnp.int32)] * 3,
        [pltpu.VMEM((CE,), jnp.int32)] * 3,
        [pltpu.VMEM((CE,), jnp.float32)] * 3,
        [pltpu.VMEM((CE, 128), jnp.float32)] * 3,
        pltpu.VMEM((8, 128), jnp.float32),
        pltpu.VMEM_SHARED((NP, 128), jnp.float32),
        pltpu.SemaphoreType.DMA,
        [pltpu.SemaphoreType.DMA] * 3,
        [pltpu.SemaphoreType.DMA] * 3,
        [pltpu.SemaphoreType.DMA] * 3,
    ),
)(_prop_body)


def _final_body(h1_ref, h2_ref, h3_ref, fd_ref, c_ref, o_ref):
    cst = c_ref[...]
    hts = []
    for k, href in enumerate((h1_ref, h2_ref, h3_ref)):
        h = href[...]
        mean = jnp.mean(h, axis=1, keepdims=True)
        var = jnp.mean(jnp.square(h - mean), axis=1, keepdims=True) \
            + jnp.float32(1e-9)
        ht = (h - mean) * cst[k:k + 1, :] * lax.rsqrt(var) \
            + cst[3 + k:4 + k, :] + cst[6 + k:7 + k, :]
        hts.append(ht)
    hop_l = cst[9:10, :]
    hop_r = cst[10:11, :]
    a_l = jnp.sum(hts[0] * hop_l, axis=1, keepdims=True)
    ls = [jnp.sum(ht * hop_r, axis=1, keepdims=True) + a_l for ht in hts]
    ls = [jnp.where(l >= 0.0, l, l * jnp.float32(0.2)) for l in ls]
    m = jnp.maximum(jnp.maximum(ls[0], ls[1]), ls[2])
    ws = [jnp.exp(l - m) for l in ls]
    tot = ws[0] + ws[1] + ws[2]
    out = fd_ref[...]
    for ht, w in zip(hts, ws):
        out = out + ht * (w / tot)
    o_ref[...] = out


def _final(h1, h2, h3, fd, consts):
    bp = 1000
    return pl.pallas_call(
        _final_body,
        grid=(N // bp,),
        in_specs=[
            pl.BlockSpec((bp, 128), lambda i: (i, 0)),
            pl.BlockSpec((bp, 128), lambda i: (i, 0)),
            pl.BlockSpec((bp, 128), lambda i: (i, 0)),
            pl.BlockSpec((bp, 128), lambda i: (i, 0)),
            pl.BlockSpec((16, 128), lambda i: (0, 0)),
        ],
        out_specs=pl.BlockSpec((bp, 128), lambda i: (i, 0)),
        out_shape=jax.ShapeDtypeStruct((N, 128), jnp.float32),
    )(h1, h2, h3, fd, consts)


def kernel(x, edge_index, W_src, W_dst, b_dst, W_attn_src, W_attn_dst,
           scale, offset, hop_attn_l, hop_attn_r, position_emb):
    srci = edge_index[0]
    dsti = edge_index[1]
    wa_pad = jnp.concatenate(
        [W_attn_src, W_attn_dst, jnp.zeros((126, 128), jnp.float32)], axis=0)
    b_row = b_dst.reshape(1, 128)

    feat_src, feat_dst, attn, cmax = _proj(x, W_src, W_dst, wa_pad, b_row)
    asrc = attn[:, 0]
    adst = attn[:, 1]
    c_off = cmax[0, 0] + cmax[0, 1]
    c_vec = jnp.full((16,), c_off, jnp.float32)

    ex, sd2, ss2 = _stats_call(asrc, adst, srci, dsti, c_vec)

    p2, q2 = _pq(sd2.reshape(NC, NP // 128, 128),
                 ss2.reshape(NC, NP // 128, 128))
    p_col = p2.reshape(NP, 1)
    q_col = q2.reshape(NP, 1)

    feat0 = jnp.pad(feat_src, ((0, NP - N), (0, 0)))
    g = _scale_rows(feat0, q_col)

    hs = []
    for _ in range(K):
        part = _prop_call(g, ex, srci, dsti)
        h, g = _merge(part, p_col, q_col)
        hs.append(h)

    consts = jnp.concatenate([
        scale[:3, 0, 0, :],
        offset[:3, 0, 0, :],
        position_emb[:, 0, :],
        hop_attn_l.reshape(1, 128),
        hop_attn_r.reshape(1, 128),
        jnp.zeros((5, 128), jnp.float32),
    ], axis=0)

    rst = _final(hs[0][:N], hs[1][:N], hs[2][:N], feat_dst, consts)
    return rst.reshape(N, 1, D)


# speedup vs baseline: 30.8081x; 1.0163x over previous
import functools

import jax
import jax.numpy as jnp
from jax import lax
from jax.experimental import pallas as pl
from jax.experimental.pallas import tpu as pltpu
from jax.experimental.pallas import tpu_sc as plsc

N = 10000
E = 320000
D = 128
K = 3
NP = 10240
NC = 2
NS = 16
NW = NC * NS
EW = E // NW
CE = 80
NCH = EW // CE
GRP = 5
NGRP = NCH // GRP
CEP = 16
NCHP = EW // CEP
NGRPP = NCHP // GRP
RPW = NP // NS

_mesh = plsc.VectorSubcoreMesh(core_axis_name="c", subcore_axis_name="s")


def _proj_body(x_ref, ws_ref, wd_ref, wa_ref, b_ref, fs_ref, fd_ref, at_ref,
               cm_ref):
    xb = x_ref[...]
    dn = (((1,), (1,)), ((), ()))
    fs_ref[...] = lax.dot_general(xb, ws_ref[...], dn,
                                  preferred_element_type=jnp.float32)
    fd_ref[...] = lax.dot_general(xb, wd_ref[...], dn,
                                  preferred_element_type=jnp.float32) + b_ref[...]
    at = lax.dot_general(xb, wa_ref[...], dn,
                         preferred_element_type=jnp.float32)
    at_ref[...] = at
    m8 = jnp.broadcast_to(jnp.max(at, axis=0, keepdims=True), (8, 128))

    @pl.when(pl.program_id(0) == 0)
    def _():
        cm_ref[...] = m8

    @pl.when(pl.program_id(0) > 0)
    def _():
        cm_ref[...] = jnp.maximum(cm_ref[...], m8)


def _proj(x, w_src, w_dst, wa_pad, b_row):
    bp = 1000
    return pl.pallas_call(
        _proj_body,
        grid=(N // bp,),
        in_specs=[
            pl.BlockSpec((bp, 128), lambda i: (i, 0)),
            pl.BlockSpec((128, 128), lambda i: (0, 0)),
            pl.BlockSpec((128, 128), lambda i: (0, 0)),
            pl.BlockSpec((128, 128), lambda i: (0, 0)),
            pl.BlockSpec((1, 128), lambda i: (0, 0)),
        ],
        out_specs=[
            pl.BlockSpec((bp, 128), lambda i: (i, 0)),
            pl.BlockSpec((bp, 128), lambda i: (i, 0)),
            pl.BlockSpec((bp, 128), lambda i: (i, 0)),
            pl.BlockSpec((8, 128), lambda i: (0, 0)),
        ],
        out_shape=[
            jax.ShapeDtypeStruct((N, 128), jnp.float32),
            jax.ShapeDtypeStruct((N, 128), jnp.float32),
            jax.ShapeDtypeStruct((N, 128), jnp.float32),
            jax.ShapeDtypeStruct((8, 128), jnp.float32),
        ],
    )(x, w_src, w_dst, wa_pad, b_row)


def _stats_body(asrc_hbm, adst_hbm, srci_hbm, dsti_hbm, cvec_hbm,
                ex_hbm, sd_hbm, ss_hbm,
                sall, dall, isrc, dsc, vas, vad, vex, cv_v, szero,
                sd_sh, ss_sh, semi, semg, semsc):
    cid = lax.axis_index("c")
    sid = lax.axis_index("s")
    wid = cid * NS + sid
    ebase = wid * EW

    pltpu.sync_copy(cvec_hbm, cv_v)
    cv = cv_v[...]

    l1 = pltpu.async_copy(srci_hbm.at[pl.ds(ebase, EW)], sall, semi)
    l2 = pltpu.async_copy(dsti_hbm.at[pl.ds(ebase, EW)], dall, semg[0])

    def _z(i, _):
        szero[pl.ds(i * 16, 16)] = jnp.zeros((16,), jnp.float32)
        return 0
    lax.fori_loop(0, RPW // 16, _z, 0)
    pltpu.sync_copy(szero, sd_sh.at[pl.ds(sid * RPW, RPW)])
    pltpu.sync_copy(szero, ss_sh.at[pl.ds(sid * RPW, RPW)])
    plsc.subcore_barrier()
    l1.wait()
    l2.wait()

    def _issue(j, b):
        cb = j * CE
        for g in range(CE // 16):
            so = pl.ds(g * 16, 16)
            bo = pl.ds(cb + g * 16, 16)
            isrc[b][so] = sall[bo]
            dsc[b][so] = dall[bo]
        d1 = pltpu.async_copy(asrc_hbm.at[isrc[b]], vas[b], semg[b])
        d2 = pltpu.async_copy(adst_hbm.at[dsc[b]], vad[b], semg[b])
        return d1, d2

    def _finish(j, b, d1, d2):
        cb = j * CE
        d1.wait()
        d2.wait()
        for g in range(CE // 16):
            so = pl.ds(g * 16, 16)
            v = vas[b][so] + vad[b][so]
            e = jnp.where(v >= 0.0, v, v * jnp.float32(0.2))
            vex[b][so] = jnp.exp(e - cv)
        pltpu.sync_copy(vex[b], ex_hbm.at[pl.ds(ebase + cb, CE)])
        s1 = pltpu.async_copy(vex[b], sd_sh.at[dsc[b]], semsc[b],
                              add=True)
        s2 = pltpu.async_copy(vex[b], ss_sh.at[isrc[b]], semsc[b],
                              add=True)
        return s1, s2

    def _pair(jj, _):
        d0 = _issue(jj * 2, 0)
        d1 = _issue(jj * 2 + 1, 1)
        s0 = _finish(jj * 2, 0, *d0)
        s1 = _finish(jj * 2 + 1, 1, *d1)
        for s in s0 + s1:
            s.wait()
        return 0

    lax.fori_loop(0, NCH // 2, _pair, 0)
    dt = _issue(NCH - 1, 0)
    for s in _finish(NCH - 1, 0, *dt):
        s.wait()
    plsc.subcore_barrier()
    sl = pl.ds(sid * RPW, RPW)
    pltpu.sync_copy(sd_sh.at[sl], sd_hbm.at[cid, sl])
    pltpu.sync_copy(ss_sh.at[sl], ss_hbm.at[cid, sl])


_stats_call = functools.partial(
    pl.kernel,
    out_type=(
        jax.ShapeDtypeStruct((E,), jnp.float32),
        jax.ShapeDtypeStruct((NC, NP), jnp.float32),
        jax.ShapeDtypeStruct((NC, NP), jnp.float32),
    ),
    mesh=_mesh,
    scratch_types=(
        pltpu.VMEM((EW,), jnp.int32),
        pltpu.VMEM((EW,), jnp.int32),
        [pltpu.VMEM((CE,), jnp.int32)] * 2,
        [pltpu.VMEM((CE,), jnp.int32)] * 2,
        [pltpu.VMEM((CE,), jnp.float32)] * 2,
        [pltpu.VMEM((CE,), jnp.float32)] * 2,
        [pltpu.VMEM((CE,), jnp.float32)] * 2,
        pltpu.VMEM((16,), jnp.float32),
        pltpu.VMEM((RPW,), jnp.float32),
        pltpu.VMEM_SHARED((NP,), jnp.float32),
        pltpu.VMEM_SHARED((NP,), jnp.float32),
        pltpu.SemaphoreType.DMA,
        [pltpu.SemaphoreType.DMA] * 2,
        [pltpu.SemaphoreType.DMA] * 2,
    ),
)(_stats_body)


def _pq_body(sd_ref, ss_ref, p_ref, q_ref):
    sd = sd_ref[0] + sd_ref[1]
    ss = ss_ref[0] + ss_ref[1]
    p_ref[...] = lax.rsqrt(jnp.maximum(sd, jnp.float32(1e-30)))
    q_ref[...] = lax.rsqrt(jnp.maximum(ss, jnp.float32(1e-30)))


def _pq(sd3, ss3):
    return pl.pallas_call(
        _pq_body,
        out_shape=[
            jax.ShapeDtypeStruct((NP // 128, 128), jnp.float32),
            jax.ShapeDtypeStruct((NP // 128, 128), jnp.float32),
        ],
    )(sd3, ss3)


def _scale_body(f_ref, s_ref, o_ref):
    o_ref[...] = f_ref[...] * s_ref[...]


def _scale_rows(feat, col):
    bp = 1024
    return pl.pallas_call(
        _scale_body,
        grid=(NP // bp,),
        in_specs=[
            pl.BlockSpec((bp, 128), lambda i: (i, 0)),
            pl.BlockSpec((bp, 1), lambda i: (i, 0)),
        ],
        out_specs=pl.BlockSpec((bp, 128), lambda i: (i, 0)),
        out_shape=jax.ShapeDtypeStruct((NP, 128), jnp.float32),
    )(feat, col)


def _merge_body(pt_ref, p_ref, q_ref, h_ref, g_ref):
    h = (pt_ref[0] + pt_ref[1]) * p_ref[...]
    h_ref[...] = h
    g_ref[...] = h * q_ref[...]


def _merge(part, p_col, q_col):
    bp = 1024
    return pl.pallas_call(
        _merge_body,
        grid=(NP // bp,),
        in_specs=[
            pl.BlockSpec((NC, bp, 128), lambda i: (0, i, 0)),
            pl.BlockSpec((bp, 1), lambda i: (i, 0)),
            pl.BlockSpec((bp, 1), lambda i: (i, 0)),
        ],
        out_specs=[
            pl.BlockSpec((bp, 128), lambda i: (i, 0)),
            pl.BlockSpec((bp, 128), lambda i: (i, 0)),
        ],
        out_shape=[
            jax.ShapeDtypeStruct((NP, 128), jnp.float32),
            jax.ShapeDtypeStruct((NP, 128), jnp.float32),
        ],
    )(part, p_col, q_col)


def _prop_body(g_hbm, w_hbm, srci_hbm, dsti_hbm, out_hbm,
               sall, isrc, dsc, wv, rows, zbuf, acc_sh,
               semi, semg, semw, semsc):
    cid = lax.axis_index("c")
    sid = lax.axis_index("s")
    wid = cid * NS + sid
    ebase = wid * EW

    l1 = pltpu.async_copy(srci_hbm.at[pl.ds(ebase, EW)], sall, semi)

    def _z(i, _):
        for t in range(8):
            zbuf[i, pl.ds(t * 16, 16)] = jnp.zeros((16,), jnp.float32)
        return 0
    lax.fori_loop(0, 8, _z, 0)

    zds = [pltpu.async_copy(zbuf,
                            acc_sh.at[pl.ds(sid * RPW + i * 8, 8), :],
                            semw[i % 3])
           for i in range(RPW // 8)]
    for d in zds:
        d.wait()
    plsc.subcore_barrier()
    l1.wait()

    def _issue(j, b):
        cb = j * CE
        for g in range(CE // 16):
            so = pl.ds(g * 16, 16)
            isrc[b][so] = sall[pl.ds(cb + g * 16, 16)]
        pltpu.async_copy(g_hbm.at[isrc[b]], rows[b], semg[b])
        pltpu.async_copy(dsti_hbm.at[pl.ds(ebase + cb, CE)], dsc[b],
                         semw[b])
        pltpu.async_copy(w_hbm.at[pl.ds(ebase + cb, CE)], wv[b],
                         semw[b])

    def _wait_in(b):
        pltpu.make_async_copy(g_hbm.at[pl.ds(0, CE), :], rows[b],
                              semg[b]).wait()
        pltpu.make_async_copy(dsti_hbm.at[pl.ds(0, CE)], dsc[b],
                              semw[b]).wait()
        pltpu.make_async_copy(w_hbm.at[pl.ds(0, CE)], wv[b],
                              semw[b]).wait()

    def _wait_sc(b):
        pltpu.make_async_copy(g_hbm.at[pl.ds(0, CE), :], rows[b],
                              semsc[b]).wait()

    def _finish(b):
        _wait_in(b)

        def _mul(g, _):
            w16 = wv[b][pl.ds(g * 16, 16)]
            for u in range(16):
                s = w16[u]
                for t in range(8):
                    sl = pl.ds(t * 16, 16)
                    rows[b][g * 16 + u, sl] = rows[b][g * 16 + u, sl] * s
            return 0
        lax.fori_loop(0, CE // 16, _mul, 0)
        pltpu.async_copy(rows[b], acc_sh.at[dsc[b]], semsc[b], add=True)

    NB = NCH // 3
    for b in range(3):
        _issue(b, b)

    def _body(i, _):
        _finish(0)
        _finish(1)
        _wait_sc(0)
        _issue(i * 3 + 3, 0)
        _finish(2)
        _wait_sc(1)
        _issue(i * 3 + 4, 1)
        _wait_sc(2)
        _issue(i * 3 + 5, 2)
        return 0

    lax.fori_loop(0, NB - 1, _body, 0)
    _finish(0)
    _finish(1)
    _wait_sc(0)
    _issue(NCH - 2, 0)
    _finish(2)
    _wait_sc(1)
    _issue(NCH - 1, 1)
    _finish(0)
    _finish(1)
    _wait_sc(2)
    _wait_sc(0)
    _wait_sc(1)

    plsc.subcore_barrier()
    sl = pl.ds(sid * RPW, RPW)
    pltpu.sync_copy(acc_sh.at[sl, :], out_hbm.at[cid, sl, :])


_prop_call = functools.partial(
    pl.kernel,
    out_type=jax.ShapeDtypeStruct((NC, NP, 128), jnp.float32),
    mesh=_mesh,
    scratch_types=(
        pltpu.VMEM((EW,), jnp.int32),
        [pltpu.VMEM((CE,), jnp.int32)] * 3,
        [pltpu.VMEM((CE,), jnp.int32)] * 3,
        [pltpu.VMEM((CE,), jnp.float32)] * 3,
        [pltpu.VMEM((CE, 128), jnp.float32)] * 3,
        pltpu.VMEM((8, 128), jnp.float32),
        pltpu.VMEM_SHARED((NP, 128), jnp.float32),
        pltpu.SemaphoreType.DMA,
        [pltpu.SemaphoreType.DMA] * 3,
        [pltpu.SemaphoreType.DMA] * 3,
        [pltpu.SemaphoreType.DMA] * 3,
    ),
)(_prop_body)


def _final_body(h1_ref, h2_ref, h3_ref, fd_ref, c_ref, o_ref):
    cst = c_ref[...]
    hts = []
    for k, href in enumerate((h1_ref, h2_ref, h3_ref)):
        h = href[...]
        mean = jnp.mean(h, axis=1, keepdims=True)
        var = jnp.mean(jnp.square(h - mean), axis=1, keepdims=True) \
            + jnp.float32(1e-9)
        ht = (h - mean) * cst[k:k + 1, :] * lax.rsqrt(var) \
            + cst[3 + k:4 + k, :] + cst[6 + k:7 + k, :]
        hts.append(ht)
    hop_l = cst[9:10, :]
    hop_r = cst[10:11, :]
    a_l = jnp.sum(hts[0] * hop_l, axis=1, keepdims=True)
    ls = [jnp.sum(ht * hop_r, axis=1, keepdims=True) + a_l for ht in hts]
    ls = [jnp.where(l >= 0.0, l, l * jnp.float32(0.2)) for l in ls]
    m = jnp.maximum(jnp.maximum(ls[0], ls[1]), ls[2])
    ws = [jnp.exp(l - m) for l in ls]
    tot = ws[0] + ws[1] + ws[2]
    out = fd_ref[...]
    for ht, w in zip(hts, ws):
        out = out + ht * (w / tot)
    o_ref[...] = out


def _final(h1, h2, h3, fd, consts):
    bp = 1000
    return pl.pallas_call(
        _final_body,
        grid=(N // bp,),
        in_specs=[
            pl.BlockSpec((bp, 128), lambda i: (i, 0)),
            pl.BlockSpec((bp, 128), lambda i: (i, 0)),
            pl.BlockSpec((bp, 128), lambda i: (i, 0)),
            pl.BlockSpec((bp, 128), lambda i: (i, 0)),
            pl.BlockSpec((16, 128), lambda i: (0, 0)),
        ],
        out_specs=pl.BlockSpec((bp, 128), lambda i: (i, 0)),
        out_shape=jax.ShapeDtypeStruct((N, 128), jnp.float32),
    )(h1, h2, h3, fd, consts)


def kernel(x, edge_index, W_src, W_dst, b_dst, W_attn_src, W_attn_dst,
           scale, offset, hop_attn_l, hop_attn_r, position_emb):
    srci = edge_index[0]
    dsti = edge_index[1]
    wa_pad = jnp.concatenate(
        [W_attn_src, W_attn_dst, jnp.zeros((126, 128), jnp.float32)], axis=0)
    b_row = b_dst.reshape(1, 128)

    feat_src, feat_dst, attn, cmax = _proj(x, W_src, W_dst, wa_pad, b_row)
    asrc = attn[:, 0]
    adst = attn[:, 1]
    c_off = cmax[0, 0] + cmax[0, 1]
    c_vec = jnp.full((16,), c_off, jnp.float32)

    ex, sd2, ss2 = _stats_call(asrc, adst, srci, dsti, c_vec)

    p2, q2 = _pq(sd2.reshape(NC, NP // 128, 128),
                 ss2.reshape(NC, NP // 128, 128))
    p_col = p2.reshape(NP, 1)
    q_col = q2.reshape(NP, 1)

    feat0 = jnp.pad(feat_src, ((0, NP - N), (0, 0)))
    g = _scale_rows(feat0, q_col)

    hs = []
    for _ in range(K):
        part = _prop_call(g, ex, srci, dsti)
        h, g = _merge(part, p_col, q_col)
        hs.append(h)

    consts = jnp.concatenate([
        scale[:3, 0, 0, :],
        offset[:3, 0, 0, :],
        position_emb[:, 0, :],
        hop_attn_l.reshape(1, 128),
        hop_attn_r.reshape(1, 128),
        jnp.zeros((5, 128), jnp.float32),
    ], axis=0)

    rst = _final(hs[0][:N], hs[1][:N], hs[2][:N], feat_dst, consts)
    return rst.reshape(N, 1, D)


# speedup vs baseline: 32.1750x; 1.0444x over previous
import functools

import jax
import jax.numpy as jnp
from jax import lax
from jax.experimental import pallas as pl
from jax.experimental.pallas import tpu as pltpu
from jax.experimental.pallas import tpu_sc as plsc

N = 10000
E = 320000
D = 128
K = 3
NP = 10240
NC = 2
NS = 16
NW = NC * NS
EW = E // NW
CE = 80
NCH = EW // CE
GRP = 5
NGRP = NCH // GRP
CEP = 16
NCHP = EW // CEP
NGRPP = NCHP // GRP
RPW = NP // NS

_mesh = plsc.VectorSubcoreMesh(core_axis_name="c", subcore_axis_name="s")


def _proj_body(x_ref, ws_ref, wd_ref, wa_ref, b_ref, fs_ref, fd_ref, at_ref,
               cm_ref):
    xb = x_ref[...]
    dn = (((1,), (1,)), ((), ()))
    fs_ref[...] = lax.dot_general(xb, ws_ref[...], dn,
                                  preferred_element_type=jnp.float32)
    fd_ref[...] = lax.dot_general(xb, wd_ref[...], dn,
                                  preferred_element_type=jnp.float32) + b_ref[...]
    at = lax.dot_general(xb, wa_ref[...], dn,
                         preferred_element_type=jnp.float32)
    at_ref[...] = at
    m8 = jnp.broadcast_to(jnp.max(at, axis=0, keepdims=True), (8, 128))

    @pl.when(pl.program_id(0) == 0)
    def _():
        cm_ref[...] = m8

    @pl.when(pl.program_id(0) > 0)
    def _():
        cm_ref[...] = jnp.maximum(cm_ref[...], m8)


def _proj(x, w_src, w_dst, wa_pad, b_row):
    bp = 1000
    return pl.pallas_call(
        _proj_body,
        grid=(N // bp,),
        in_specs=[
            pl.BlockSpec((bp, 128), lambda i: (i, 0)),
            pl.BlockSpec((128, 128), lambda i: (0, 0)),
            pl.BlockSpec((128, 128), lambda i: (0, 0)),
            pl.BlockSpec((128, 128), lambda i: (0, 0)),
            pl.BlockSpec((1, 128), lambda i: (0, 0)),
        ],
        out_specs=[
            pl.BlockSpec((bp, 128), lambda i: (i, 0)),
            pl.BlockSpec((bp, 128), lambda i: (i, 0)),
            pl.BlockSpec((bp, 128), lambda i: (i, 0)),
            pl.BlockSpec((8, 128), lambda i: (0, 0)),
        ],
        out_shape=[
            jax.ShapeDtypeStruct((N, 128), jnp.float32),
            jax.ShapeDtypeStruct((N, 128), jnp.float32),
            jax.ShapeDtypeStruct((N, 128), jnp.float32),
            jax.ShapeDtypeStruct((8, 128), jnp.float32),
        ],
    )(x, w_src, w_dst, wa_pad, b_row)


def _stats_body(asrc_hbm, adst_hbm, srci_hbm, dsti_hbm, cvec_hbm,
                ex_hbm, sd_hbm, ss_hbm,
                sall, dall, isrc, dsc, vas, vad, vex, cv_v, szero,
                sd_sh, ss_sh, semi, semg, semsc):
    cid = lax.axis_index("c")
    sid = lax.axis_index("s")
    wid = cid * NS + sid
    ebase = wid * EW

    pltpu.sync_copy(cvec_hbm, cv_v)
    cv = cv_v[...]

    l1 = pltpu.async_copy(srci_hbm.at[pl.ds(ebase, EW)], sall, semi)
    l2 = pltpu.async_copy(dsti_hbm.at[pl.ds(ebase, EW)], dall, semg[0])

    def _z(i, _):
        szero[pl.ds(i * 16, 16)] = jnp.zeros((16,), jnp.float32)
        return 0
    lax.fori_loop(0, RPW // 16, _z, 0)
    pltpu.sync_copy(szero, sd_sh.at[pl.ds(sid * RPW, RPW)])
    pltpu.sync_copy(szero, ss_sh.at[pl.ds(sid * RPW, RPW)])
    plsc.subcore_barrier()
    l1.wait()
    l2.wait()

    def _issue(j, b):
        cb = j * CE
        for g in range(CE // 16):
            so = pl.ds(g * 16, 16)
            bo = pl.ds(cb + g * 16, 16)
            isrc[b][so] = sall[bo]
            dsc[b][so] = dall[bo]
        pltpu.async_copy(asrc_hbm.at[isrc[b]], vas[b], semg[b])
        pltpu.async_copy(adst_hbm.at[dsc[b]], vad[b], semg[b])

    def _wait_sc(b):
        pltpu.make_async_copy(ex_hbm.at[pl.ds(0, CE)], vex[b],
                              semsc[b]).wait()
        pltpu.make_async_copy(ex_hbm.at[pl.ds(0, CE)], vex[b],
                              semsc[b]).wait()

    def _finish(j, b):
        cb = j * CE
        pltpu.make_async_copy(asrc_hbm.at[pl.ds(0, CE)], vas[b],
                              semg[b]).wait()
        pltpu.make_async_copy(adst_hbm.at[pl.ds(0, CE)], vad[b],
                              semg[b]).wait()
        for g in range(CE // 16):
            so = pl.ds(g * 16, 16)
            v = vas[b][so] + vad[b][so]
            e = jnp.where(v >= 0.0, v, v * jnp.float32(0.2))
            vex[b][so] = jnp.exp(e - cv)
        pltpu.sync_copy(vex[b], ex_hbm.at[pl.ds(ebase + cb, CE)])
        pltpu.async_copy(vex[b], sd_sh.at[dsc[b]], semsc[b], add=True)
        pltpu.async_copy(vex[b], ss_sh.at[isrc[b]], semsc[b], add=True)

    for b in range(3):
        _issue(b, b)

    def _body(i, _):
        _finish(i * 3, 0)
        _finish(i * 3 + 1, 1)
        _wait_sc(0)
        _issue(i * 3 + 3, 0)
        _finish(i * 3 + 2, 2)
        _wait_sc(1)
        _issue(i * 3 + 4, 1)
        _wait_sc(2)
        _issue(i * 3 + 5, 2)
        return 0

    NB = NCH // 3
    lax.fori_loop(0, NB - 1, _body, 0)
    base_t = (NB - 1) * 3
    _finish(base_t, 0)
    _finish(base_t + 1, 1)
    _wait_sc(0)
    _issue(NCH - 2, 0)
    _finish(base_t + 2, 2)
    _wait_sc(1)
    _issue(NCH - 1, 1)
    _finish(NCH - 2, 0)
    _finish(NCH - 1, 1)
    _wait_sc(2)
    _wait_sc(0)
    _wait_sc(1)
    plsc.subcore_barrier()
    sl = pl.ds(sid * RPW, RPW)
    pltpu.sync_copy(sd_sh.at[sl], sd_hbm.at[cid, sl])
    pltpu.sync_copy(ss_sh.at[sl], ss_hbm.at[cid, sl])


_stats_call = functools.partial(
    pl.kernel,
    out_type=(
        jax.ShapeDtypeStruct((E,), jnp.float32),
        jax.ShapeDtypeStruct((NC, NP), jnp.float32),
        jax.ShapeDtypeStruct((NC, NP), jnp.float32),
    ),
    mesh=_mesh,
    scratch_types=(
        pltpu.VMEM((EW,), jnp.int32),
        pltpu.VMEM((EW,), jnp.int32),
        [pltpu.VMEM((CE,), jnp.int32)] * 3,
        [pltpu.VMEM((CE,), jnp.int32)] * 3,
        [pltpu.VMEM((CE,), jnp.float32)] * 3,
        [pltpu.VMEM((CE,), jnp.float32)] * 3,
        [pltpu.VMEM((CE,), jnp.float32)] * 3,
        pltpu.VMEM((16,), jnp.float32),
        pltpu.VMEM((RPW,), jnp.float32),
        pltpu.VMEM_SHARED((NP,), jnp.float32),
        pltpu.VMEM_SHARED((NP,), jnp.float32),
        pltpu.SemaphoreType.DMA,
        [pltpu.SemaphoreType.DMA] * 3,
        [pltpu.SemaphoreType.DMA] * 3,
    ),
)(_stats_body)


def _pq_body(sd_ref, ss_ref, p_ref, q_ref):
    sd = sd_ref[0] + sd_ref[1]
    ss = ss_ref[0] + ss_ref[1]
    p_ref[...] = lax.rsqrt(jnp.maximum(sd, jnp.float32(1e-30)))
    q_ref[...] = lax.rsqrt(jnp.maximum(ss, jnp.float32(1e-30)))


def _pq(sd3, ss3):
    return pl.pallas_call(
        _pq_body,
        out_shape=[
            jax.ShapeDtypeStruct((NP // 128, 128), jnp.float32),
            jax.ShapeDtypeStruct((NP // 128, 128), jnp.float32),
        ],
    )(sd3, ss3)


def _scale_body(f_ref, s_ref, o_ref):
    o_ref[...] = f_ref[...] * s_ref[...]


def _scale_rows(feat, col):
    bp = 1024
    return pl.pallas_call(
        _scale_body,
        grid=(NP // bp,),
        in_specs=[
            pl.BlockSpec((bp, 128), lambda i: (i, 0)),
            pl.BlockSpec((bp, 1), lambda i: (i, 0)),
        ],
        out_specs=pl.BlockSpec((bp, 128), lambda i: (i, 0)),
        out_shape=jax.ShapeDtypeStruct((NP, 128), jnp.float32),
    )(feat, col)


def _merge_body(pt_ref, p_ref, q_ref, h_ref, g_ref):
    h = (pt_ref[0] + pt_ref[1]) * p_ref[...]
    h_ref[...] = h
    g_ref[...] = h * q_ref[...]


def _merge(part, p_col, q_col):
    bp = 1024
    return pl.pallas_call(
        _merge_body,
        grid=(NP // bp,),
        in_specs=[
            pl.BlockSpec((NC, bp, 128), lambda i: (0, i, 0)),
            pl.BlockSpec((bp, 1), lambda i: (i, 0)),
            pl.BlockSpec((bp, 1), lambda i: (i, 0)),
        ],
        out_specs=[
            pl.BlockSpec((bp, 128), lambda i: (i, 0)),
            pl.BlockSpec((bp, 128), lambda i: (i, 0)),
        ],
        out_shape=[
            jax.ShapeDtypeStruct((NP, 128), jnp.float32),
            jax.ShapeDtypeStruct((NP, 128), jnp.float32),
        ],
    )(part, p_col, q_col)


def _prop_body(g_hbm, w_hbm, srci_hbm, dsti_hbm, out_hbm,
               sall, isrc, dsc, wv, rows, zbuf, acc_sh,
               semi, semg, semw, semsc):
    cid = lax.axis_index("c")
    sid = lax.axis_index("s")
    wid = cid * NS + sid
    ebase = wid * EW

    l1 = pltpu.async_copy(srci_hbm.at[pl.ds(ebase, EW)], sall, semi)

    def _z(i, _):
        for t in range(8):
            zbuf[i, pl.ds(t * 16, 16)] = jnp.zeros((16,), jnp.float32)
        return 0
    lax.fori_loop(0, 8, _z, 0)

    zds = [pltpu.async_copy(zbuf,
                            acc_sh.at[pl.ds(sid * RPW + i * 8, 8), :],
                            semw[i % 3])
           for i in range(RPW // 8)]
    for d in zds:
        d.wait()
    plsc.subcore_barrier()
    l1.wait()

    def _issue(j, b):
        cb = j * CE
        for g in range(CE // 16):
            so = pl.ds(g * 16, 16)
            isrc[b][so] = sall[pl.ds(cb + g * 16, 16)]
        pltpu.async_copy(g_hbm.at[isrc[b]], rows[b], semg[b])
        pltpu.async_copy(dsti_hbm.at[pl.ds(ebase + cb, CE)], dsc[b],
                         semw[b])
        pltpu.async_copy(w_hbm.at[pl.ds(ebase + cb, CE)], wv[b],
                         semw[b])

    def _wait_in(b):
        pltpu.make_async_copy(g_hbm.at[pl.ds(0, CE), :], rows[b],
                              semg[b]).wait()
        pltpu.make_async_copy(dsti_hbm.at[pl.ds(0, CE)], dsc[b],
                              semw[b]).wait()
        pltpu.make_async_copy(w_hbm.at[pl.ds(0, CE)], wv[b],
                              semw[b]).wait()

    def _wait_sc(b):
        pltpu.make_async_copy(g_hbm.at[pl.ds(0, CE), :], rows[b],
                              semsc[b]).wait()

    def _finish(b):
        _wait_in(b)

        def _mul(g, _):
            w16 = wv[b][pl.ds(g * 16, 16)]
            for u in range(16):
                s = w16[u]
                for t in range(8):
                    sl = pl.ds(t * 16, 16)
                    rows[b][g * 16 + u, sl] = rows[b][g * 16 + u, sl] * s
            return 0
        lax.fori_loop(0, CE // 16, _mul, 0)
        pltpu.async_copy(rows[b], acc_sh.at[dsc[b]], semsc[b], add=True)

    NB = NCH // 3
    for b in range(3):
        _issue(b, b)

    def _body(i, _):
        _finish(0)
        _finish(1)
        _wait_sc(0)
        _issue(i * 3 + 3, 0)
        _finish(2)
        _wait_sc(1)
        _issue(i * 3 + 4, 1)
        _wait_sc(2)
        _issue(i * 3 + 5, 2)
        return 0

    lax.fori_loop(0, NB - 1, _body, 0)
    _finish(0)
    _finish(1)
    _wait_sc(0)
    _issue(NCH - 2, 0)
    _finish(2)
    _wait_sc(1)
    _issue(NCH - 1, 1)
    _finish(0)
    _finish(1)
    _wait_sc(2)
    _wait_sc(0)
    _wait_sc(1)

    plsc.subcore_barrier()
    sl = pl.ds(sid * RPW, RPW)
    pltpu.sync_copy(acc_sh.at[sl, :], out_hbm.at[cid, sl, :])


_prop_call = functools.partial(
    pl.kernel,
    out_type=jax.ShapeDtypeStruct((NC, NP, 128), jnp.float32),
    mesh=_mesh,
    scratch_types=(
        pltpu.VMEM((EW,), jnp.int32),
        [pltpu.VMEM((CE,), jnp.int32)] * 3,
        [pltpu.VMEM((CE,), jnp.int32)] * 3,
        [pltpu.VMEM((CE,), jnp.float32)] * 3,
        [pltpu.VMEM((CE, 128), jnp.float32)] * 3,
        pltpu.VMEM((8, 128), jnp.float32),
        pltpu.VMEM_SHARED((NP, 128), jnp.float32),
        pltpu.SemaphoreType.DMA,
        [pltpu.SemaphoreType.DMA] * 3,
        [pltpu.SemaphoreType.DMA] * 3,
        [pltpu.SemaphoreType.DMA] * 3,
    ),
)(_prop_body)


def _final_body(h1_ref, h2_ref, h3_ref, fd_ref, c_ref, o_ref):
    cst = c_ref[...]
    hts = []
    for k, href in enumerate((h1_ref, h2_ref, h3_ref)):
        h = href[...]
        mean = jnp.mean(h, axis=1, keepdims=True)
        var = jnp.mean(jnp.square(h - mean), axis=1, keepdims=True) \
            + jnp.float32(1e-9)
        ht = (h - mean) * cst[k:k + 1, :] * lax.rsqrt(var) \
            + cst[3 + k:4 + k, :] + cst[6 + k:7 + k, :]
        hts.append(ht)
    hop_l = cst[9:10, :]
    hop_r = cst[10:11, :]
    a_l = jnp.sum(hts[0] * hop_l, axis=1, keepdims=True)
    ls = [jnp.sum(ht * hop_r, axis=1, keepdims=True) + a_l for ht in hts]
    ls = [jnp.where(l >= 0.0, l, l * jnp.float32(0.2)) for l in ls]
    m = jnp.maximum(jnp.maximum(ls[0], ls[1]), ls[2])
    ws = [jnp.exp(l - m) for l in ls]
    tot = ws[0] + ws[1] + ws[2]
    out = fd_ref[...]
    for ht, w in zip(hts, ws):
        out = out + ht * (w / tot)
    o_ref[...] = out


def _final(h1, h2, h3, fd, consts):
    bp = 1000
    return pl.pallas_call(
        _final_body,
        grid=(N // bp,),
        in_specs=[
            pl.BlockSpec((bp, 128), lambda i: (i, 0)),
            pl.BlockSpec((bp, 128), lambda i: (i, 0)),
            pl.BlockSpec((bp, 128), lambda i: (i, 0)),
            pl.BlockSpec((bp, 128), lambda i: (i, 0)),
            pl.BlockSpec((16, 128), lambda i: (0, 0)),
        ],
        out_specs=pl.BlockSpec((bp, 128), lambda i: (i, 0)),
        out_shape=jax.ShapeDtypeStruct((N, 128), jnp.float32),
    )(h1, h2, h3, fd, consts)


def kernel(x, edge_index, W_src, W_dst, b_dst, W_attn_src, W_attn_dst,
           scale, offset, hop_attn_l, hop_attn_r, position_emb):
    srci = edge_index[0]
    dsti = edge_index[1]
    wa_pad = jnp.concatenate(
        [W_attn_src, W_attn_dst, jnp.zeros((126, 128), jnp.float32)], axis=0)
    b_row = b_dst.reshape(1, 128)

    feat_src, feat_dst, attn, cmax = _proj(x, W_src, W_dst, wa_pad, b_row)
    asrc = attn[:, 0]
    adst = attn[:, 1]
    c_off = cmax[0, 0] + cmax[0, 1]
    c_vec = jnp.full((16,), c_off, jnp.float32)

    ex, sd2, ss2 = _stats_call(asrc, adst, srci, dsti, c_vec)

    p2, q2 = _pq(sd2.reshape(NC, NP // 128, 128),
                 ss2.reshape(NC, NP // 128, 128))
    p_col = p2.reshape(NP, 1)
    q_col = q2.reshape(NP, 1)

    feat0 = jnp.pad(feat_src, ((0, NP - N), (0, 0)))
    g = _scale_rows(feat0, q_col)

    hs = []
    for _ in range(K):
        part = _prop_call(g, ex, srci, dsti)
        h, g = _merge(part, p_col, q_col)
        hs.append(h)

    consts = jnp.concatenate([
        scale[:3, 0, 0, :],
        offset[:3, 0, 0, :],
        position_emb[:, 0, :],
        hop_attn_l.reshape(1, 128),
        hop_attn_r.reshape(1, 128),
        jnp.zeros((5, 128), jnp.float32),
    ], axis=0)

    rst = _final(hs[0][:N], hs[1][:N], hs[2][:N], feat_dst, consts)
    return rst.reshape(N, 1, D)
